# Initial kernel scaffold; baseline (speedup 1.0000x reference)
import jax
import jax.numpy as jnp
from jax.experimental import pallas as pl


def kernel(feat_srl, feat_tok, bert_token_emb, edge_index_rel, span_start, span_len, edge_index_s2t, rel_W, rel_b, nt_W1, nt_b1, nt_W2, nt_b2, att_W, att_b, gru_Wih, gru_Whh, gru_bih, gru_bhh):
    raise NotImplementedError("write your pallas kernel here")


# speedup vs baseline: 4.6836x; 4.6836x over previous
import functools

import jax
import jax.numpy as jnp
from jax import lax
from jax.experimental import pallas as pl
from jax.experimental.pallas import tpu as pltpu
from jax.experimental.pallas import tpu_sc as plsc

IN = 128
OUT = 128
BERT = 768
N_SRL = 10000
N_TOK = 10000
E_REL = 160000
E_S2T = 160000
T = 512
NSTART = 500
NLEN = 8
NSPAN = NLEN * NSTART

NC = 2
NS = 16
NW = NC * NS
CH = 128
E_PAD = 163840
EPW = E_PAD // NW
NCH = EPW // CH
EPT = E_PAD // NS
NCH2 = EPT // CH
ACC_N = 10240
ZPT = ACC_N // NS
CPA = 632
CPL = N_SRL - 15 * CPA

_f32 = jnp.float32


def _leaky(x):
    return jnp.where(x >= 0, x, 0.01 * x)


def _prep_body(bert_ref, fs_ref, ft_ref, rwa_ref, rwb_ref, rb_ref,
               w1_ref, b1_ref, w2_ref, b2_ref, aw2_ref, ab_ref,
               a_ref, r_ref, batt_ref, htok_ref):
    bert_w = jnp.dot(bert_ref[...], rwb_ref[...], preferred_element_type=_f32)
    t_col = lax.broadcasted_iota(jnp.int32, (NSTART, T), 1)
    s_row = lax.broadcasted_iota(jnp.int32, (NSTART, T), 0)
    for l in range(NLEN):
        band = jnp.where((t_col >= s_row) & (t_col <= s_row + l),
                         _f32(1.0 / (l + 1)), _f32(0.0))
        r_ref[l] = jnp.dot(band, bert_w, preferred_element_type=_f32)

    def ntrans(x):
        u = jnp.dot(x, w1_ref[...], preferred_element_type=_f32) + b1_ref[...]
        u = _leaky(u)
        return jnp.dot(u, w2_ref[...], preferred_element_type=_f32) + b2_ref[...]

    fs = fs_ref[...]
    a_ref[...] = jnp.dot(fs, rwa_ref[...], preferred_element_type=_f32) + rb_ref[...]
    t = ntrans(fs)
    batt_ref[...] = jnp.dot(t, aw2_ref[...], preferred_element_type=_f32) + ab_ref[...]
    htok_ref[...] = ntrans(ft_ref[...])


def _tc_prep(bert, fs, ft, rwa, rwb, rb, w1, b1, w2, b2, aw2, ab):
    return pl.pallas_call(
        _prep_body,
        out_shape=(
            jax.ShapeDtypeStruct((N_SRL, IN), _f32),
            jax.ShapeDtypeStruct((NLEN, NSTART, OUT), _f32),
            jax.ShapeDtypeStruct((N_SRL, OUT), _f32),
            jax.ShapeDtypeStruct((N_TOK, OUT), _f32),
        ),
    )(bert, fs, ft, rwa, rwb, rb, w1, b1, w2, b2, aw2, ab)


_MESH = plsc.VectorSubcoreMesh(core_axis_name="c", subcore_axis_name="s",
                               num_cores=NC, num_subcores=NS)


def _sc_gather(a_tab, r_tab, b_tab, src_g, sid_g, dst_g):
    @functools.partial(
        pl.kernel,
        out_type=(
            jax.ShapeDtypeStruct((E_PAD, OUT), _f32),
            jax.ShapeDtypeStruct((E_PAD, OUT), _f32),
            jax.ShapeDtypeStruct((E_PAD, OUT), _f32),
        ),
        mesh=_MESH,
        scratch_types=[
            pltpu.VMEM((NCH, CH), jnp.int32),
            pltpu.VMEM((NCH, CH), jnp.int32),
            pltpu.VMEM((NCH, CH), jnp.int32),
            pltpu.VMEM((CH, OUT), _f32),
            pltpu.VMEM((CH, OUT), _f32),
            pltpu.VMEM((CH, OUT), _f32),
            pltpu.SemaphoreType.DMA,
            pltpu.SemaphoreType.DMA,
            pltpu.SemaphoreType.DMA,
        ],
    )
    def k(a_h, r_h, b_h, src_h, sid_h, dst_h, oa, orr, ob,
          src_v, sid_v, dst_v, ra, rr, rb_, m1, m2, m3):
        c = lax.axis_index("c")
        s = lax.axis_index("s")
        wid = s * NC + c
        base = wid * EPW
        pltpu.sync_copy(src_h.at[wid], src_v)
        pltpu.sync_copy(sid_h.at[wid], sid_v)
        pltpu.sync_copy(dst_h.at[wid], dst_v)

        def chunk(j, carry):
            c1 = pltpu.async_copy(a_h.at[src_v.at[j]], ra, m1)
            c2 = pltpu.async_copy(r_h.at[sid_v.at[j]], rr, m2)
            c3 = pltpu.async_copy(b_h.at[dst_v.at[j]], rb_, m3)
            c1.wait()
            c2.wait()
            c3.wait()
            off = base + j * CH
            pltpu.sync_copy(ra, oa.at[pl.ds(off, CH)])
            pltpu.sync_copy(rr, orr.at[pl.ds(off, CH)])
            pltpu.sync_copy(rb_, ob.at[pl.ds(off, CH)])
            return carry

        lax.fori_loop(0, NCH, chunk, 0)

    return k(a_tab, r_tab, b_tab, src_g, sid_g, dst_g)


_BE = 2048


def _edge_body(a_ref, r_ref, b_ref, w1_ref, b1_ref, w2_ref, b2_ref, aw1_ref,
               ex_ref, p_ref):
    pre = a_ref[...] + r_ref[...]
    u = jnp.dot(pre, w1_ref[...], preferred_element_type=_f32) + b1_ref[...]
    u = _leaky(u)
    m = jnp.dot(u, w2_ref[...], preferred_element_type=_f32) + b2_ref[...]
    e = jnp.dot(m, aw1_ref[...], preferred_element_type=_f32) + b_ref[...]
    e = _leaky(e)
    ex = jnp.exp(e)
    ex_ref[...] = ex
    p_ref[...] = ex * m


def _tc_edge(a_src, r_e, b_dst, w1, b1, w2, b2, aw1):
    nblk = E_PAD // _BE
    edge_spec = pl.BlockSpec((_BE, OUT), lambda i: (i, 0))
    w_spec = pl.BlockSpec((IN, OUT), lambda i: (0, 0))
    bias_spec = pl.BlockSpec((1, OUT), lambda i: (0, 0))
    return pl.pallas_call(
        _edge_body,
        grid=(nblk,),
        in_specs=[edge_spec, edge_spec, edge_spec,
                  w_spec, bias_spec, w_spec, bias_spec, w_spec],
        out_specs=[edge_spec, edge_spec],
        out_shape=(
            jax.ShapeDtypeStruct((E_PAD, OUT), _f32),
            jax.ShapeDtypeStruct((E_PAD, OUT), _f32),
        ),
    )(a_src, r_e, b_dst, w1, b1, w2, b2, aw1)


def _sc_scatter(ex, p, dst_s, zrows):
    @functools.partial(
        pl.kernel,
        out_type=(
            jax.ShapeDtypeStruct((N_SRL, OUT), _f32),
            jax.ShapeDtypeStruct((N_SRL, OUT), _f32),
        ),
        mesh=_MESH,
        scratch_types=[
            pltpu.VMEM((NCH2, CH), jnp.int32),
            pltpu.VMEM((CH, OUT), _f32),
            pltpu.VMEM_SHARED((ACC_N, OUT), _f32),
        ],
    )
    def k(ex_h, p_h, dst_h, z_h, den_o, num_o, idx_v, rows_v, acc):
        c = lax.axis_index("c")
        s = lax.axis_index("s")
        pltpu.sync_copy(z_h, acc.at[pl.ds(s * ZPT, ZPT)])
        pltpu.sync_copy(dst_h.at[s], idx_v)
        plsc.subcore_barrier()

        @pl.when(c == 0)
        def _():
            def chunk(j, carry):
                pltpu.sync_copy(ex_h.at[pl.ds(s * EPT + j * CH, CH)], rows_v)
                pltpu.sync_copy(rows_v, acc.at[idx_v.at[j]], add=True)
                return carry
            lax.fori_loop(0, NCH2, chunk, 0)

        @pl.when(c == 1)
        def _():
            def chunk(j, carry):
                pltpu.sync_copy(p_h.at[pl.ds(s * EPT + j * CH, CH)], rows_v)
                pltpu.sync_copy(rows_v, acc.at[idx_v.at[j]], add=True)
                return carry
            lax.fori_loop(0, NCH2, chunk, 0)

        plsc.subcore_barrier()

        @pl.when((c == 0) & (s < NS - 1))
        def _():
            pltpu.sync_copy(acc.at[pl.ds(s * CPA, CPA)],
                            den_o.at[pl.ds(s * CPA, CPA)])

        @pl.when((c == 0) & (s == NS - 1))
        def _():
            pltpu.sync_copy(acc.at[pl.ds((NS - 1) * CPA, CPL)],
                            den_o.at[pl.ds((NS - 1) * CPA, CPL)])

        @pl.when((c == 1) & (s < NS - 1))
        def _():
            pltpu.sync_copy(acc.at[pl.ds(s * CPA, CPA)],
                            num_o.at[pl.ds(s * CPA, CPA)])

        @pl.when((c == 1) & (s == NS - 1))
        def _():
            pltpu.sync_copy(acc.at[pl.ds((NS - 1) * CPA, CPL)],
                            num_o.at[pl.ds((NS - 1) * CPA, CPL)])

    return k(ex, p, dst_s, zrows)


def _fin_body(den_ref, num_ref, fs_ref, out_ref):
    den = den_ref[...]
    keep = den > 0
    safe = jnp.where(keep, den, _f32(1.0))
    out_ref[...] = jnp.where(keep, num_ref[...] / safe, fs_ref[...])


def _tc_fin(denom, numer, fs):
    return pl.pallas_call(
        _fin_body,
        out_shape=jax.ShapeDtypeStruct((N_SRL, IN), _f32),
    )(denom, numer, fs)


def _sc_s2t(h_srl, s2_g, d2_g, zrows):
    @functools.partial(
        pl.kernel,
        out_type=jax.ShapeDtypeStruct((NC, N_TOK, OUT), _f32),
        mesh=_MESH,
        scratch_types=[
            pltpu.VMEM((NCH, CH), jnp.int32),
            pltpu.VMEM((NCH, CH), jnp.int32),
            pltpu.VMEM((CH, OUT), _f32),
            pltpu.VMEM_SHARED((ACC_N, OUT), _f32),
            pltpu.SemaphoreType.DMA,
        ],
    )
    def k(h_h, s2_h, d2_h, z_h, out_o, s2_v, d2_v, rows_v, acc, sem):
        c = lax.axis_index("c")
        s = lax.axis_index("s")
        wid = s * NC + c
        pltpu.sync_copy(z_h, acc.at[pl.ds(s * ZPT, ZPT)])
        pltpu.sync_copy(s2_h.at[wid], s2_v)
        pltpu.sync_copy(d2_h.at[wid], d2_v)
        plsc.subcore_barrier()

        def chunk(j, carry):
            pltpu.async_copy(h_h.at[s2_v.at[j]], rows_v, sem).wait()
            pltpu.sync_copy(rows_v, acc.at[d2_v.at[j]], add=True)
            return carry

        lax.fori_loop(0, NCH, chunk, 0)
        plsc.subcore_barrier()

        @pl.when(s < NS - 1)
        def _():
            pltpu.sync_copy(acc.at[pl.ds(s * CPA, CPA)],
                            out_o.at[c, pl.ds(s * CPA, CPA)])

        @pl.when(s == NS - 1)
        def _():
            pltpu.sync_copy(acc.at[pl.ds((NS - 1) * CPA, CPL)],
                            out_o.at[c, pl.ds((NS - 1) * CPA, CPL)])

    return k(h_srl, s2_g, d2_g, zrows)


def _gru_body(p0_ref, p1_ref, htok_ref, wih_ref, whh_ref, bih_ref, bhh_ref,
              out_ref):
    x1 = p0_ref[...] + p1_ref[...]
    h = jnp.zeros((N_TOK, OUT), _f32)
    for x in (x1, htok_ref[...]):
        gi = jnp.dot(x, wih_ref[...], preferred_element_type=_f32) + bih_ref[...]
        gh = jnp.dot(h, whh_ref[...], preferred_element_type=_f32) + bhh_ref[...]
        ir, iz, inn = gi[:, :OUT], gi[:, OUT:2 * OUT], gi[:, 2 * OUT:]
        hr, hz, hn = gh[:, :OUT], gh[:, OUT:2 * OUT], gh[:, 2 * OUT:]
        r = jax.nn.sigmoid(ir + hr)
        z = jax.nn.sigmoid(iz + hz)
        n = jnp.tanh(inn + r * hn)
        h = (1.0 - z) * n + z * h
    out_ref[...] = h


def _tc_gru(p0, p1, htok, wih, whh, bih, bhh):
    return pl.pallas_call(
        _gru_body,
        out_shape=jax.ShapeDtypeStruct((N_TOK, OUT), _f32),
    )(p0, p1, htok, wih, whh, bih, bhh)


def kernel(feat_srl, feat_tok, bert_token_emb, edge_index_rel, span_start,
           span_len, edge_index_s2t, rel_W, rel_b, nt_W1, nt_b1, nt_W2, nt_b2,
           att_W, att_b, gru_Wih, gru_Whh, gru_bih, gru_bhh):
    rwa = rel_W[:IN]
    rwb = rel_W[IN:]
    aw1 = att_W[:OUT]
    aw2 = att_W[OUT:]
    rb = rel_b.reshape(1, IN)
    b1 = nt_b1.reshape(1, IN)
    b2 = nt_b2.reshape(1, OUT)
    ab = att_b.reshape(1, OUT)
    bih = gru_bih.reshape(1, 3 * OUT)
    bhh = gru_bhh.reshape(1, 3 * OUT)

    a_tab, r_tab3, b_tab, h_tok = _tc_prep(
        bert_token_emb, feat_srl, feat_tok, rwa, rwb, rb,
        nt_W1, b1, nt_W2, b2, aw2, ab)
    r_tab = r_tab3.reshape(NSPAN, OUT)

    npad = E_PAD - E_REL
    zpad = jnp.zeros((npad,), jnp.int32)
    trash = jnp.full((npad,), N_SRL, jnp.int32)
    src = jnp.concatenate([edge_index_rel[0].astype(jnp.int32), zpad])
    dst = jnp.concatenate([edge_index_rel[1].astype(jnp.int32), trash])
    sid = jnp.concatenate(
        [span_len.astype(jnp.int32) * NSTART + span_start.astype(jnp.int32),
         zpad])
    src_g = src.reshape(NW, NCH, CH)
    sid_g = sid.reshape(NW, NCH, CH)
    dst_g = jnp.where(dst >= N_SRL, 0, dst).reshape(NW, NCH, CH)

    a_src, r_e, b_dst = _sc_gather(a_tab, r_tab, b_tab, src_g, sid_g, dst_g)

    ex, p = _tc_edge(a_src, r_e, b_dst, nt_W1, b1, nt_W2, b2, aw1)

    zrows = jnp.zeros((ZPT, OUT), _f32)
    dst_s = dst.reshape(NS, NCH2, CH)
    denom, numer = _sc_scatter(ex, p, dst_s, zrows)

    h_srl = _tc_fin(denom, numer, feat_srl)

    s2_g = jnp.concatenate(
        [edge_index_s2t[0].astype(jnp.int32), zpad]).reshape(NW, NCH, CH)
    d2_g = jnp.concatenate(
        [edge_index_s2t[1].astype(jnp.int32), trash]).reshape(NW, NCH, CH)
    partials = _sc_s2t(h_srl, s2_g, d2_g, zrows)

    h_out = _tc_gru(partials[0], partials[1], h_tok, gru_Wih, gru_Whh, bih, bhh)
    return (h_srl, h_out)


# speedup vs baseline: 4.7061x; 1.0048x over previous
import functools

import jax
import jax.numpy as jnp
from jax import lax
from jax.experimental import pallas as pl
from jax.experimental.pallas import tpu as pltpu
from jax.experimental.pallas import tpu_sc as plsc

IN = 128
OUT = 128
BERT = 768
N_SRL = 10000
N_TOK = 10000
E_REL = 160000
E_S2T = 160000
T = 512
NSTART = 500
NLEN = 8
NSPAN = NLEN * NSTART

NC = 2
NS = 16
NW = NC * NS
CH = 128
E_PAD = 163840
EPW = E_PAD // NW
NCH = EPW // CH
EPT = E_PAD // NS
NCH2 = EPT // CH
ACC_N = 10240
ZPT = ACC_N // NS
CPA = 632
CPL = N_SRL - 15 * CPA

_f32 = jnp.float32


def _leaky(x):
    return jnp.where(x >= 0, x, 0.01 * x)


def _prep_body(bert_ref, fs_ref, ft_ref, rwa_ref, rwb_ref, rb_ref,
               w1_ref, b1_ref, w2_ref, b2_ref, aw2_ref, ab_ref,
               a_ref, r_ref, batt_ref, htok_ref):
    bert_w = jnp.dot(bert_ref[...], rwb_ref[...], preferred_element_type=_f32)
    t_col = lax.broadcasted_iota(jnp.int32, (NSTART, T), 1)
    s_row = lax.broadcasted_iota(jnp.int32, (NSTART, T), 0)
    for l in range(NLEN):
        band = jnp.where((t_col >= s_row) & (t_col <= s_row + l),
                         _f32(1.0 / (l + 1)), _f32(0.0))
        r_ref[l] = jnp.dot(band, bert_w, preferred_element_type=_f32)

    def ntrans(x):
        u = jnp.dot(x, w1_ref[...], preferred_element_type=_f32) + b1_ref[...]
        u = _leaky(u)
        return jnp.dot(u, w2_ref[...], preferred_element_type=_f32) + b2_ref[...]

    fs = fs_ref[...]
    a_ref[...] = jnp.dot(fs, rwa_ref[...], preferred_element_type=_f32) + rb_ref[...]
    t = ntrans(fs)
    batt_ref[...] = jnp.dot(t, aw2_ref[...], preferred_element_type=_f32) + ab_ref[...]
    htok_ref[...] = ntrans(ft_ref[...])


def _tc_prep(bert, fs, ft, rwa, rwb, rb, w1, b1, w2, b2, aw2, ab):
    return pl.pallas_call(
        _prep_body,
        out_shape=(
            jax.ShapeDtypeStruct((N_SRL, IN), _f32),
            jax.ShapeDtypeStruct((NLEN, NSTART, OUT), _f32),
            jax.ShapeDtypeStruct((N_SRL, OUT), _f32),
            jax.ShapeDtypeStruct((N_TOK, OUT), _f32),
        ),
    )(bert, fs, ft, rwa, rwb, rb, w1, b1, w2, b2, aw2, ab)


_MESH = plsc.VectorSubcoreMesh(core_axis_name="c", subcore_axis_name="s",
                               num_cores=NC, num_subcores=NS)


def _sc_gather(a_tab, r_tab, b_tab, src_g, sid_g, dst_g):
    @functools.partial(
        pl.kernel,
        out_type=(
            jax.ShapeDtypeStruct((E_PAD, OUT), _f32),
            jax.ShapeDtypeStruct((E_PAD, OUT), _f32),
            jax.ShapeDtypeStruct((E_PAD, OUT), _f32),
        ),
        mesh=_MESH,
        scratch_types=[
            pltpu.VMEM((NCH, CH), jnp.int32),
            pltpu.VMEM((NCH, CH), jnp.int32),
            pltpu.VMEM((NCH, CH), jnp.int32),
            pltpu.VMEM((CH, OUT), _f32),
            pltpu.VMEM((CH, OUT), _f32),
            pltpu.VMEM((CH, OUT), _f32),
            pltpu.VMEM((CH, OUT), _f32),
            pltpu.VMEM((CH, OUT), _f32),
            pltpu.VMEM((CH, OUT), _f32),
            pltpu.SemaphoreType.DMA,
            pltpu.SemaphoreType.DMA,
        ],
    )
    def k(a_h, r_h, b_h, src_h, sid_h, dst_h, oa, orr, ob,
          src_v, sid_v, dst_v, ra0, rr0, rb0, ra1, rr1, rb1, g0, g1):
        c = lax.axis_index("c")
        s = lax.axis_index("s")
        wid = s * NC + c
        base = wid * EPW
        pltpu.sync_copy(src_h.at[wid], src_v)
        pltpu.sync_copy(sid_h.at[wid], sid_v)
        pltpu.sync_copy(dst_h.at[wid], dst_v)

        set0 = (ra0, rr0, rb0)
        set1 = (ra1, rr1, rb1)

        def issue(j, bufs, sem):
            pltpu.async_copy(a_h.at[src_v.at[j]], bufs[0], sem)
            pltpu.async_copy(r_h.at[sid_v.at[j]], bufs[1], sem)
            pltpu.async_copy(b_h.at[dst_v.at[j]], bufs[2], sem)

        def drain(bufs, sem):
            for buf in bufs:
                pltpu.make_async_copy(a_h.at[src_v.at[0]], buf, sem).wait()

        def write(j, bufs):
            off = base + j * CH
            pltpu.sync_copy(bufs[0], oa.at[pl.ds(off, CH)])
            pltpu.sync_copy(bufs[1], orr.at[pl.ds(off, CH)])
            pltpu.sync_copy(bufs[2], ob.at[pl.ds(off, CH)])

        issue(0, set0, g0)

        def body(k_, carry):
            j0 = 2 * k_
            j1 = j0 + 1
            drain(set0, g0)
            issue(j1, set1, g1)
            write(j0, set0)
            drain(set1, g1)
            issue(jnp.minimum(j0 + 2, NCH - 1), set0, g0)
            write(j1, set1)
            return carry

        lax.fori_loop(0, NCH // 2, body, 0)
        drain(set0, g0)

    return k(a_tab, r_tab, b_tab, src_g, sid_g, dst_g)


_BE = 2048


def _edge_body(a_ref, r_ref, b_ref, w1_ref, b1_ref, w2_ref, b2_ref, aw1_ref,
               ex_ref, p_ref):
    pre = a_ref[...] + r_ref[...]
    u = jnp.dot(pre, w1_ref[...], preferred_element_type=_f32) + b1_ref[...]
    u = _leaky(u)
    m = jnp.dot(u, w2_ref[...], preferred_element_type=_f32) + b2_ref[...]
    e = jnp.dot(m, aw1_ref[...], preferred_element_type=_f32) + b_ref[...]
    e = _leaky(e)
    ex = jnp.exp(e)
    ex_ref[...] = ex
    p_ref[...] = ex * m


def _tc_edge(a_src, r_e, b_dst, w1, b1, w2, b2, aw1):
    nblk = E_PAD // _BE
    edge_spec = pl.BlockSpec((_BE, OUT), lambda i: (i, 0))
    w_spec = pl.BlockSpec((IN, OUT), lambda i: (0, 0))
    bias_spec = pl.BlockSpec((1, OUT), lambda i: (0, 0))
    return pl.pallas_call(
        _edge_body,
        grid=(nblk,),
        in_specs=[edge_spec, edge_spec, edge_spec,
                  w_spec, bias_spec, w_spec, bias_spec, w_spec],
        out_specs=[edge_spec, edge_spec],
        out_shape=(
            jax.ShapeDtypeStruct((E_PAD, OUT), _f32),
            jax.ShapeDtypeStruct((E_PAD, OUT), _f32),
        ),
    )(a_src, r_e, b_dst, w1, b1, w2, b2, aw1)


def _sc_scatter(ex, p, dst_s, zrows):
    @functools.partial(
        pl.kernel,
        out_type=(
            jax.ShapeDtypeStruct((N_SRL, OUT), _f32),
            jax.ShapeDtypeStruct((N_SRL, OUT), _f32),
        ),
        mesh=_MESH,
        scratch_types=[
            pltpu.VMEM((NCH2, CH), jnp.int32),
            pltpu.VMEM((CH, OUT), _f32),
            pltpu.VMEM((CH, OUT), _f32),
            pltpu.VMEM_SHARED((ACC_N, OUT), _f32),
            pltpu.SemaphoreType.DMA,
            pltpu.SemaphoreType.DMA,
        ],
    )
    def k(ex_h, p_h, dst_h, z_h, den_o, num_o, idx_v, rows0, rows1, acc,
          r0, r1):
        c = lax.axis_index("c")
        s = lax.axis_index("s")
        pltpu.sync_copy(z_h, acc.at[pl.ds(s * ZPT, ZPT)])
        pltpu.sync_copy(dst_h.at[s], idx_v)
        plsc.subcore_barrier()

        def run(src_h):
            def read(j, buf, sem):
                pltpu.async_copy(src_h.at[pl.ds(s * EPT + j * CH, CH)],
                                 buf, sem)

            def drain(buf, sem):
                pltpu.make_async_copy(src_h.at[pl.ds(0, CH)], buf, sem).wait()

            def scat(j, buf):
                pltpu.sync_copy(buf, acc.at[idx_v.at[j]], add=True)

            read(0, rows0, r0)

            def body(k_, carry):
                j0 = 2 * k_
                j1 = j0 + 1
                drain(rows0, r0)
                read(j1, rows1, r1)
                scat(j0, rows0)
                drain(rows1, r1)
                read(jnp.minimum(j0 + 2, NCH2 - 1), rows0, r0)
                scat(j1, rows1)
                return carry

            lax.fori_loop(0, NCH2 // 2, body, 0)
            drain(rows0, r0)

        @pl.when(c == 0)
        def _():
            run(ex_h)

        @pl.when(c == 1)
        def _():
            run(p_h)

        plsc.subcore_barrier()

        @pl.when((c == 0) & (s < NS - 1))
        def _():
            pltpu.sync_copy(acc.at[pl.ds(s * CPA, CPA)],
                            den_o.at[pl.ds(s * CPA, CPA)])

        @pl.when((c == 0) & (s == NS - 1))
        def _():
            pltpu.sync_copy(acc.at[pl.ds((NS - 1) * CPA, CPL)],
                            den_o.at[pl.ds((NS - 1) * CPA, CPL)])

        @pl.when((c == 1) & (s < NS - 1))
        def _():
            pltpu.sync_copy(acc.at[pl.ds(s * CPA, CPA)],
                            num_o.at[pl.ds(s * CPA, CPA)])

        @pl.when((c == 1) & (s == NS - 1))
        def _():
            pltpu.sync_copy(acc.at[pl.ds((NS - 1) * CPA, CPL)],
                            num_o.at[pl.ds((NS - 1) * CPA, CPL)])

    return k(ex, p, dst_s, zrows)


def _fin_body(den_ref, num_ref, fs_ref, out_ref):
    den = den_ref[...]
    keep = den > 0
    safe = jnp.where(keep, den, _f32(1.0))
    out_ref[...] = jnp.where(keep, num_ref[...] / safe, fs_ref[...])


def _tc_fin(denom, numer, fs):
    return pl.pallas_call(
        _fin_body,
        out_shape=jax.ShapeDtypeStruct((N_SRL, IN), _f32),
    )(denom, numer, fs)


def _sc_s2t(h_srl, s2_g, d2_g, zrows):
    @functools.partial(
        pl.kernel,
        out_type=jax.ShapeDtypeStruct((NC, N_TOK, OUT), _f32),
        mesh=_MESH,
        scratch_types=[
            pltpu.VMEM((NCH, CH), jnp.int32),
            pltpu.VMEM((NCH, CH), jnp.int32),
            pltpu.VMEM((CH, OUT), _f32),
            pltpu.VMEM((CH, OUT), _f32),
            pltpu.VMEM_SHARED((ACC_N, OUT), _f32),
            pltpu.SemaphoreType.DMA,
            pltpu.SemaphoreType.DMA,
        ],
    )
    def k(h_h, s2_h, d2_h, z_h, out_o, s2_v, d2_v, rows0, rows1, acc, g0, g1):
        c = lax.axis_index("c")
        s = lax.axis_index("s")
        wid = s * NC + c
        pltpu.sync_copy(z_h, acc.at[pl.ds(s * ZPT, ZPT)])
        pltpu.sync_copy(s2_h.at[wid], s2_v)
        pltpu.sync_copy(d2_h.at[wid], d2_v)
        plsc.subcore_barrier()

        def gat(j, buf, sem):
            pltpu.async_copy(h_h.at[s2_v.at[j]], buf, sem)

        def drain(buf, sem):
            pltpu.make_async_copy(h_h.at[s2_v.at[0]], buf, sem).wait()

        def scat(j, buf):
            pltpu.sync_copy(buf, acc.at[d2_v.at[j]], add=True)

        gat(0, rows0, g0)

        def body(k_, carry):
            j0 = 2 * k_
            j1 = j0 + 1
            drain(rows0, g0)
            gat(j1, rows1, g1)
            scat(j0, rows0)
            drain(rows1, g1)
            gat(jnp.minimum(j0 + 2, NCH - 1), rows0, g0)
            scat(j1, rows1)
            return carry

        lax.fori_loop(0, NCH // 2, body, 0)
        drain(rows0, g0)
        plsc.subcore_barrier()

        @pl.when(s < NS - 1)
        def _():
            pltpu.sync_copy(acc.at[pl.ds(s * CPA, CPA)],
                            out_o.at[c, pl.ds(s * CPA, CPA)])

        @pl.when(s == NS - 1)
        def _():
            pltpu.sync_copy(acc.at[pl.ds((NS - 1) * CPA, CPL)],
                            out_o.at[c, pl.ds((NS - 1) * CPA, CPL)])

    return k(h_srl, s2_g, d2_g, zrows)


def _gru_body(p0_ref, p1_ref, htok_ref, wih_ref, whh_ref, bih_ref, bhh_ref,
              out_ref):
    x1 = p0_ref[...] + p1_ref[...]
    h = jnp.zeros((N_TOK, OUT), _f32)
    for x in (x1, htok_ref[...]):
        gi = jnp.dot(x, wih_ref[...], preferred_element_type=_f32) + bih_ref[...]
        gh = jnp.dot(h, whh_ref[...], preferred_element_type=_f32) + bhh_ref[...]
        ir, iz, inn = gi[:, :OUT], gi[:, OUT:2 * OUT], gi[:, 2 * OUT:]
        hr, hz, hn = gh[:, :OUT], gh[:, OUT:2 * OUT], gh[:, 2 * OUT:]
        r = jax.nn.sigmoid(ir + hr)
        z = jax.nn.sigmoid(iz + hz)
        n = jnp.tanh(inn + r * hn)
        h = (1.0 - z) * n + z * h
    out_ref[...] = h


def _tc_gru(p0, p1, htok, wih, whh, bih, bhh):
    return pl.pallas_call(
        _gru_body,
        out_shape=jax.ShapeDtypeStruct((N_TOK, OUT), _f32),
    )(p0, p1, htok, wih, whh, bih, bhh)


def kernel(feat_srl, feat_tok, bert_token_emb, edge_index_rel, span_start,
           span_len, edge_index_s2t, rel_W, rel_b, nt_W1, nt_b1, nt_W2, nt_b2,
           att_W, att_b, gru_Wih, gru_Whh, gru_bih, gru_bhh):
    rwa = rel_W[:IN]
    rwb = rel_W[IN:]
    aw1 = att_W[:OUT]
    aw2 = att_W[OUT:]
    rb = rel_b.reshape(1, IN)
    b1 = nt_b1.reshape(1, IN)
    b2 = nt_b2.reshape(1, OUT)
    ab = att_b.reshape(1, OUT)
    bih = gru_bih.reshape(1, 3 * OUT)
    bhh = gru_bhh.reshape(1, 3 * OUT)

    a_tab, r_tab3, b_tab, h_tok = _tc_prep(
        bert_token_emb, feat_srl, feat_tok, rwa, rwb, rb,
        nt_W1, b1, nt_W2, b2, aw2, ab)
    r_tab = r_tab3.reshape(NSPAN, OUT)

    npad = E_PAD - E_REL
    zpad = jnp.zeros((npad,), jnp.int32)
    trash = jnp.full((npad,), N_SRL, jnp.int32)
    src = jnp.concatenate([edge_index_rel[0].astype(jnp.int32), zpad])
    dst = jnp.concatenate([edge_index_rel[1].astype(jnp.int32), trash])
    sid = jnp.concatenate(
        [span_len.astype(jnp.int32) * NSTART + span_start.astype(jnp.int32),
         zpad])
    src_g = src.reshape(NW, NCH, CH)
    sid_g = sid.reshape(NW, NCH, CH)
    dst_g = jnp.where(dst >= N_SRL, 0, dst).reshape(NW, NCH, CH)

    a_src, r_e, b_dst = _sc_gather(a_tab, r_tab, b_tab, src_g, sid_g, dst_g)

    ex, p = _tc_edge(a_src, r_e, b_dst, nt_W1, b1, nt_W2, b2, aw1)

    zrows = jnp.zeros((ZPT, OUT), _f32)
    dst_s = dst.reshape(NS, NCH2, CH)
    denom, numer = _sc_scatter(ex, p, dst_s, zrows)

    h_srl = _tc_fin(denom, numer, feat_srl)

    s2_g = jnp.concatenate(
        [edge_index_s2t[0].astype(jnp.int32), zpad]).reshape(NW, NCH, CH)
    d2_g = jnp.concatenate(
        [edge_index_s2t[1].astype(jnp.int32), trash]).reshape(NW, NCH, CH)
    partials = _sc_s2t(h_srl, s2_g, d2_g, zrows)

    h_out = _tc_gru(partials[0], partials[1], h_tok, gru_Wih, gru_Whh, bih, bhh)
    return (h_srl, h_out)


# speedup vs baseline: 4.8839x; 1.0378x over previous
import functools

import jax
import jax.numpy as jnp
from jax import lax
from jax.experimental import pallas as pl
from jax.experimental.pallas import tpu as pltpu
from jax.experimental.pallas import tpu_sc as plsc

IN = 128
OUT = 128
BERT = 768
N_SRL = 10000
N_TOK = 10000
E_REL = 160000
E_S2T = 160000
T = 512
NSTART = 500
NLEN = 8
NSPAN = NLEN * NSTART

NC = 2
NS = 16
NW = NC * NS
CH = 128
E_PAD = 163840
EPW = E_PAD // NW
NCH = EPW // CH
EPT = E_PAD // NS
NCH2 = EPT // CH
ACC_N = 10240
ZPT = ACC_N // NS
CPA = 632
CPL = N_SRL - 15 * CPA

NCHT = E_PAD // CH
K0 = 56
K1 = (NCHT // NS) - K0
CB1 = NS * K0

_f32 = jnp.float32


def _leaky(x):
    return jnp.where(x >= 0, x, 0.01 * x)


def _prep_body(bert_ref, fs_ref, ft_ref, rwa_ref, rwb_ref, rb_ref,
               w1_ref, b1_ref, w2_ref, b2_ref, aw2_ref, ab_ref,
               a_ref, r_ref, batt_ref, htok_ref):
    bert_w = jnp.dot(bert_ref[...], rwb_ref[...], preferred_element_type=_f32)
    t_col = lax.broadcasted_iota(jnp.int32, (NSTART, T), 1)
    s_row = lax.broadcasted_iota(jnp.int32, (NSTART, T), 0)
    for l in range(NLEN):
        band = jnp.where((t_col >= s_row) & (t_col <= s_row + l),
                         _f32(1.0 / (l + 1)), _f32(0.0))
        r_ref[l] = jnp.dot(band, bert_w, preferred_element_type=_f32)

    def ntrans(x):
        u = jnp.dot(x, w1_ref[...], preferred_element_type=_f32) + b1_ref[...]
        u = _leaky(u)
        return jnp.dot(u, w2_ref[...], preferred_element_type=_f32) + b2_ref[...]

    fs = fs_ref[...]
    a_ref[...] = jnp.dot(fs, rwa_ref[...], preferred_element_type=_f32) + rb_ref[...]
    t = ntrans(fs)
    batt_ref[...] = jnp.dot(t, aw2_ref[...], preferred_element_type=_f32) + ab_ref[...]
    htok_ref[...] = ntrans(ft_ref[...])


def _tc_prep(bert, fs, ft, rwa, rwb, rb, w1, b1, w2, b2, aw2, ab):
    return pl.pallas_call(
        _prep_body,
        out_shape=(
            jax.ShapeDtypeStruct((N_SRL, IN), _f32),
            jax.ShapeDtypeStruct((NLEN, NSTART, OUT), _f32),
            jax.ShapeDtypeStruct((N_SRL, OUT), _f32),
            jax.ShapeDtypeStruct((N_TOK, OUT), _f32),
        ),
    )(bert, fs, ft, rwa, rwb, rb, w1, b1, w2, b2, aw2, ab)


_MESH = plsc.VectorSubcoreMesh(core_axis_name="c", subcore_axis_name="s",
                               num_cores=NC, num_subcores=NS)


def _sc_gather(a_tab, r_tab, b_tab, src_g, sid_g, dst_g):
    @functools.partial(
        pl.kernel,
        out_type=(
            jax.ShapeDtypeStruct((E_PAD, OUT), _f32),
            jax.ShapeDtypeStruct((E_PAD, OUT), _f32),
            jax.ShapeDtypeStruct((E_PAD, OUT), _f32),
        ),
        mesh=_MESH,
        scratch_types=[
            pltpu.VMEM((K0, CH), jnp.int32),
            pltpu.VMEM((K0, CH), jnp.int32),
            pltpu.VMEM((K0, CH), jnp.int32),
            pltpu.VMEM((CH, OUT), _f32),
            pltpu.VMEM((CH, OUT), _f32),
            pltpu.VMEM((CH, OUT), _f32),
            pltpu.VMEM((CH, OUT), _f32),
            pltpu.VMEM((CH, OUT), _f32),
            pltpu.VMEM((CH, OUT), _f32),
            pltpu.SemaphoreType.DMA,
            pltpu.SemaphoreType.DMA,
        ],
    )
    def k(a_h, r_h, b_h, src_h, sid_h, dst_h, oa, orr, ob,
          src_v, sid_v, dst_v, ra0, rr0, rb0, ra1, rr1, rb1, g0, g1):
        c = lax.axis_index("c")
        s = lax.axis_index("s")

        set0 = (ra0, rr0, rb0)
        set1 = (ra1, rr1, rb1)

        def pipe(kk, cbase):
            gbase = cbase + s * kk
            pltpu.sync_copy(src_h.at[pl.ds(gbase, kk)], src_v.at[pl.ds(0, kk)])
            pltpu.sync_copy(sid_h.at[pl.ds(gbase, kk)], sid_v.at[pl.ds(0, kk)])
            pltpu.sync_copy(dst_h.at[pl.ds(gbase, kk)], dst_v.at[pl.ds(0, kk)])

            def issue(j, bufs, sem):
                pltpu.async_copy(a_h.at[src_v.at[j]], bufs[0], sem)
                pltpu.async_copy(r_h.at[sid_v.at[j]], bufs[1], sem)
                pltpu.async_copy(b_h.at[dst_v.at[j]], bufs[2], sem)

            def drain(bufs, sem):
                for buf in bufs:
                    pltpu.make_async_copy(a_h.at[src_v.at[0]], buf, sem).wait()

            def write(j, bufs):
                off = (gbase + j) * CH
                pltpu.sync_copy(bufs[0], oa.at[pl.ds(off, CH)])
                pltpu.sync_copy(bufs[1], orr.at[pl.ds(off, CH)])
                pltpu.sync_copy(bufs[2], ob.at[pl.ds(off, CH)])

            issue(0, set0, g0)

            def body(k_, carry):
                j0 = 2 * k_
                j1 = j0 + 1
                drain(set0, g0)
                issue(j1, set1, g1)
                write(j0, set0)
                drain(set1, g1)
                issue(jnp.minimum(j0 + 2, kk - 1), set0, g0)
                write(j1, set1)
                return carry

            lax.fori_loop(0, kk // 2, body, 0)
            drain(set0, g0)

        @pl.when(c == 0)
        def _():
            pipe(K0, 0)

        @pl.when(c == 1)
        def _():
            pipe(K1, CB1)

    return k(a_tab, r_tab, b_tab, src_g, sid_g, dst_g)


_BE = 2048


def _edge_body(a_ref, r_ref, b_ref, w1_ref, b1_ref, w2_ref, b2_ref, aw1_ref,
               ex_ref, p_ref):
    pre = a_ref[...] + r_ref[...]
    u = jnp.dot(pre, w1_ref[...], preferred_element_type=_f32) + b1_ref[...]
    u = _leaky(u)
    m = jnp.dot(u, w2_ref[...], preferred_element_type=_f32) + b2_ref[...]
    e = jnp.dot(m, aw1_ref[...], preferred_element_type=_f32) + b_ref[...]
    e = _leaky(e)
    ex = jnp.exp(e)
    ex_ref[...] = ex
    p_ref[...] = ex * m


def _tc_edge(a_src, r_e, b_dst, w1, b1, w2, b2, aw1):
    nblk = E_PAD // _BE
    edge_spec = pl.BlockSpec((_BE, OUT), lambda i: (i, 0))
    w_spec = pl.BlockSpec((IN, OUT), lambda i: (0, 0))
    bias_spec = pl.BlockSpec((1, OUT), lambda i: (0, 0))
    return pl.pallas_call(
        _edge_body,
        grid=(nblk,),
        in_specs=[edge_spec, edge_spec, edge_spec,
                  w_spec, bias_spec, w_spec, bias_spec, w_spec],
        out_specs=[edge_spec, edge_spec],
        out_shape=(
            jax.ShapeDtypeStruct((E_PAD, OUT), _f32),
            jax.ShapeDtypeStruct((E_PAD, OUT), _f32),
        ),
    )(a_src, r_e, b_dst, w1, b1, w2, b2, aw1)


def _sc_scatter(ex, p, dst_s, zrows):
    @functools.partial(
        pl.kernel,
        out_type=(
            jax.ShapeDtypeStruct((N_SRL, OUT), _f32),
            jax.ShapeDtypeStruct((N_SRL, OUT), _f32),
        ),
        mesh=_MESH,
        scratch_types=[
            pltpu.VMEM((NCH2, CH), jnp.int32),
            pltpu.VMEM((CH, OUT), _f32),
            pltpu.VMEM((CH, OUT), _f32),
            pltpu.VMEM_SHARED((ACC_N, OUT), _f32),
            pltpu.SemaphoreType.DMA,
            pltpu.SemaphoreType.DMA,
        ],
    )
    def k(ex_h, p_h, dst_h, z_h, den_o, num_o, idx_v, rows0, rows1, acc,
          r0, r1):
        c = lax.axis_index("c")
        s = lax.axis_index("s")
        pltpu.sync_copy(z_h, acc.at[pl.ds(s * ZPT, ZPT)])
        pltpu.sync_copy(dst_h.at[s], idx_v)
        plsc.subcore_barrier()

        def run(src_h):
            def read(j, buf, sem):
                pltpu.async_copy(src_h.at[pl.ds(s * EPT + j * CH, CH)],
                                 buf, sem)

            def drain(buf, sem):
                pltpu.make_async_copy(src_h.at[pl.ds(0, CH)], buf, sem).wait()

            def scat(j, buf):
                pltpu.sync_copy(buf, acc.at[idx_v.at[j]], add=True)

            read(0, rows0, r0)

            def body(k_, carry):
                j0 = 2 * k_
                j1 = j0 + 1
                drain(rows0, r0)
                read(j1, rows1, r1)
                scat(j0, rows0)
                drain(rows1, r1)
                read(jnp.minimum(j0 + 2, NCH2 - 1), rows0, r0)
                scat(j1, rows1)
                return carry

            lax.fori_loop(0, NCH2 // 2, body, 0)
            drain(rows0, r0)

        @pl.when(c == 0)
        def _():
            run(ex_h)

        @pl.when(c == 1)
        def _():
            run(p_h)

        plsc.subcore_barrier()

        @pl.when((c == 0) & (s < NS - 1))
        def _():
            pltpu.sync_copy(acc.at[pl.ds(s * CPA, CPA)],
                            den_o.at[pl.ds(s * CPA, CPA)])

        @pl.when((c == 0) & (s == NS - 1))
        def _():
            pltpu.sync_copy(acc.at[pl.ds((NS - 1) * CPA, CPL)],
                            den_o.at[pl.ds((NS - 1) * CPA, CPL)])

        @pl.when((c == 1) & (s < NS - 1))
        def _():
            pltpu.sync_copy(acc.at[pl.ds(s * CPA, CPA)],
                            num_o.at[pl.ds(s * CPA, CPA)])

        @pl.when((c == 1) & (s == NS - 1))
        def _():
            pltpu.sync_copy(acc.at[pl.ds((NS - 1) * CPA, CPL)],
                            num_o.at[pl.ds((NS - 1) * CPA, CPL)])

    return k(ex, p, dst_s, zrows)


def _fin_body(den_ref, num_ref, fs_ref, out_ref):
    den = den_ref[...]
    keep = den > 0
    safe = jnp.where(keep, den, _f32(1.0))
    out_ref[...] = jnp.where(keep, num_ref[...] / safe, fs_ref[...])


def _tc_fin(denom, numer, fs):
    return pl.pallas_call(
        _fin_body,
        out_shape=jax.ShapeDtypeStruct((N_SRL, IN), _f32),
    )(denom, numer, fs)


def _sc_s2t(h_srl, s2_g, d2_g, zrows):
    @functools.partial(
        pl.kernel,
        out_type=jax.ShapeDtypeStruct((NC, N_TOK, OUT), _f32),
        mesh=_MESH,
        scratch_types=[
            pltpu.VMEM((K0, CH), jnp.int32),
            pltpu.VMEM((K0, CH), jnp.int32),
            pltpu.VMEM((CH, OUT), _f32),
            pltpu.VMEM((CH, OUT), _f32),
            pltpu.VMEM_SHARED((ACC_N, OUT), _f32),
            pltpu.SemaphoreType.DMA,
            pltpu.SemaphoreType.DMA,
        ],
    )
    def k(h_h, s2_h, d2_h, z_h, out_o, s2_v, d2_v, rows0, rows1, acc, g0, g1):
        c = lax.axis_index("c")
        s = lax.axis_index("s")
        pltpu.sync_copy(z_h, acc.at[pl.ds(s * ZPT, ZPT)])
        plsc.subcore_barrier()

        def pipe(kk, cbase):
            gbase = cbase + s * kk
            pltpu.sync_copy(s2_h.at[pl.ds(gbase, kk)], s2_v.at[pl.ds(0, kk)])
            pltpu.sync_copy(d2_h.at[pl.ds(gbase, kk)], d2_v.at[pl.ds(0, kk)])

            def gat(j, buf, sem):
                pltpu.async_copy(h_h.at[s2_v.at[j]], buf, sem)

            def drain(buf, sem):
                pltpu.make_async_copy(h_h.at[s2_v.at[0]], buf, sem).wait()

            def scat(j, buf):
                pltpu.sync_copy(buf, acc.at[d2_v.at[j]], add=True)

            gat(0, rows0, g0)

            def body(k_, carry):
                j0 = 2 * k_
                j1 = j0 + 1
                drain(rows0, g0)
                gat(j1, rows1, g1)
                scat(j0, rows0)
                drain(rows1, g1)
                gat(jnp.minimum(j0 + 2, kk - 1), rows0, g0)
                scat(j1, rows1)
                return carry

            lax.fori_loop(0, kk // 2, body, 0)
            drain(rows0, g0)

        @pl.when(c == 0)
        def _():
            pipe(K0, 0)

        @pl.when(c == 1)
        def _():
            pipe(K1, CB1)

        plsc.subcore_barrier()

        @pl.when(s < NS - 1)
        def _():
            pltpu.sync_copy(acc.at[pl.ds(s * CPA, CPA)],
                            out_o.at[c, pl.ds(s * CPA, CPA)])

        @pl.when(s == NS - 1)
        def _():
            pltpu.sync_copy(acc.at[pl.ds((NS - 1) * CPA, CPL)],
                            out_o.at[c, pl.ds((NS - 1) * CPA, CPL)])

    return k(h_srl, s2_g, d2_g, zrows)


def _gru_body(p0_ref, p1_ref, htok_ref, wih_ref, whh_ref, bih_ref, bhh_ref,
              out_ref):
    x1 = p0_ref[...] + p1_ref[...]
    h = jnp.zeros((N_TOK, OUT), _f32)
    for x in (x1, htok_ref[...]):
        gi = jnp.dot(x, wih_ref[...], preferred_element_type=_f32) + bih_ref[...]
        gh = jnp.dot(h, whh_ref[...], preferred_element_type=_f32) + bhh_ref[...]
        ir, iz, inn = gi[:, :OUT], gi[:, OUT:2 * OUT], gi[:, 2 * OUT:]
        hr, hz, hn = gh[:, :OUT], gh[:, OUT:2 * OUT], gh[:, 2 * OUT:]
        r = jax.nn.sigmoid(ir + hr)
        z = jax.nn.sigmoid(iz + hz)
        n = jnp.tanh(inn + r * hn)
        h = (1.0 - z) * n + z * h
    out_ref[...] = h


def _tc_gru(p0, p1, htok, wih, whh, bih, bhh):
    return pl.pallas_call(
        _gru_body,
        out_shape=jax.ShapeDtypeStruct((N_TOK, OUT), _f32),
    )(p0, p1, htok, wih, whh, bih, bhh)


def kernel(feat_srl, feat_tok, bert_token_emb, edge_index_rel, span_start,
           span_len, edge_index_s2t, rel_W, rel_b, nt_W1, nt_b1, nt_W2, nt_b2,
           att_W, att_b, gru_Wih, gru_Whh, gru_bih, gru_bhh):
    rwa = rel_W[:IN]
    rwb = rel_W[IN:]
    aw1 = att_W[:OUT]
    aw2 = att_W[OUT:]
    rb = rel_b.reshape(1, IN)
    b1 = nt_b1.reshape(1, IN)
    b2 = nt_b2.reshape(1, OUT)
    ab = att_b.reshape(1, OUT)
    bih = gru_bih.reshape(1, 3 * OUT)
    bhh = gru_bhh.reshape(1, 3 * OUT)

    a_tab, r_tab3, b_tab, h_tok = _tc_prep(
        bert_token_emb, feat_srl, feat_tok, rwa, rwb, rb,
        nt_W1, b1, nt_W2, b2, aw2, ab)
    r_tab = r_tab3.reshape(NSPAN, OUT)

    npad = E_PAD - E_REL
    zpad = jnp.zeros((npad,), jnp.int32)
    trash = jnp.full((npad,), N_SRL, jnp.int32)
    src = jnp.concatenate([edge_index_rel[0].astype(jnp.int32), zpad])
    dst = jnp.concatenate([edge_index_rel[1].astype(jnp.int32), trash])
    sid = jnp.concatenate(
        [span_len.astype(jnp.int32) * NSTART + span_start.astype(jnp.int32),
         zpad])
    src_g = src.reshape(NCHT, CH)
    sid_g = sid.reshape(NCHT, CH)
    dst_g = jnp.where(dst >= N_SRL, 0, dst).reshape(NCHT, CH)

    a_src, r_e, b_dst = _sc_gather(a_tab, r_tab, b_tab, src_g, sid_g, dst_g)

    ex, p = _tc_edge(a_src, r_e, b_dst, nt_W1, b1, nt_W2, b2, aw1)

    zrows = jnp.zeros((ZPT, OUT), _f32)
    dst_s = dst.reshape(NS, NCH2, CH)
    denom, numer = _sc_scatter(ex, p, dst_s, zrows)

    h_srl = _tc_fin(denom, numer, feat_srl)

    s2_g = jnp.concatenate(
        [edge_index_s2t[0].astype(jnp.int32), zpad]).reshape(NCHT, CH)
    d2_g = jnp.concatenate(
        [edge_index_s2t[1].astype(jnp.int32), trash]).reshape(NCHT, CH)
    partials = _sc_s2t(h_srl, s2_g, d2_g, zrows)

    h_out = _tc_gru(partials[0], partials[1], h_tok, gru_Wih, gru_Whh, bih, bhh)
    return (h_srl, h_out)


# speedup vs baseline: 7.6480x; 1.5660x over previous
import functools

import jax
import jax.numpy as jnp
from jax import lax
from jax.experimental import pallas as pl
from jax.experimental.pallas import tpu as pltpu
from jax.experimental.pallas import tpu_sc as plsc

IN = 128
OUT = 128
BERT = 768
N_SRL = 10000
N_TOK = 10000
E_REL = 160000
E_S2T = 160000
T = 512
NSTART = 500
NLEN = 8
NSPAN = NLEN * NSTART

NC = 2
NS = 16
NW = NC * NS
CH = 128
E_PAD = 163840
EPW = E_PAD // NW
NCH = EPW // CH
EPT = E_PAD // NS
NCH2 = EPT // CH
ACC_N = 10240
ZPT = ACC_N // NS
CPA = 632
CPL = N_SRL - 15 * CPA

NCHT = E_PAD // CH
K0 = 56
K1 = (NCHT // NS) - K0
CB1 = NS * K0
NCHPT = NCHT // NS
GP = 520

_f32 = jnp.float32


def _leaky(x):
    return jnp.where(x >= 0, x, 0.01 * x)


def _prep_body(bert_ref, fs_ref, ft_ref, rwa_ref, rwb_ref, rb_ref,
               w1_ref, b1_ref, w2_ref, b2_ref, aw2_ref, ab_ref,
               a_ref, r_ref, batt_ref, htok_ref):
    bert_w = jnp.dot(bert_ref[...], rwb_ref[...], preferred_element_type=_f32)
    k_row = lax.broadcasted_iota(jnp.int32, (GP, T), 0)
    t_col = lax.broadcasted_iota(jnp.int32, (GP, T), 1)
    lmat = jnp.where(t_col < k_row, _f32(1.0), _f32(0.0))
    r_ref[...] = jnp.dot(lmat, bert_w, preferred_element_type=_f32)

    def ntrans(x):
        u = jnp.dot(x, w1_ref[...], preferred_element_type=_f32) + b1_ref[...]
        u = _leaky(u)
        return jnp.dot(u, w2_ref[...], preferred_element_type=_f32) + b2_ref[...]

    fs = fs_ref[...]
    a_ref[...] = jnp.dot(fs, rwa_ref[...], preferred_element_type=_f32) + rb_ref[...]
    t = ntrans(fs)
    batt_ref[...] = jnp.dot(t, aw2_ref[...], preferred_element_type=_f32) + ab_ref[...]
    htok_ref[...] = ntrans(ft_ref[...])


def _tc_prep(bert, fs, ft, rwa, rwb, rb, w1, b1, w2, b2, aw2, ab):
    return pl.pallas_call(
        _prep_body,
        out_shape=(
            jax.ShapeDtypeStruct((N_SRL, IN), _f32),
            jax.ShapeDtypeStruct((GP, OUT), _f32),
            jax.ShapeDtypeStruct((N_SRL, OUT), _f32),
            jax.ShapeDtypeStruct((N_TOK, OUT), _f32),
        ),
    )(bert, fs, ft, rwa, rwb, rb, w1, b1, w2, b2, aw2, ab)


_MESH = plsc.VectorSubcoreMesh(core_axis_name="c", subcore_axis_name="s",
                               num_cores=NC, num_subcores=NS)


def _sc_gather(a_tab, b_tab, src_g, dst_g):
    @functools.partial(
        pl.kernel,
        out_type=(
            jax.ShapeDtypeStruct((E_PAD, OUT), _f32),
            jax.ShapeDtypeStruct((E_PAD, OUT), _f32),
        ),
        mesh=_MESH,
        scratch_types=[
            pltpu.VMEM((NCHPT, CH), jnp.int32),
            pltpu.VMEM((CH, OUT), _f32),
            pltpu.VMEM((CH, OUT), _f32),
            pltpu.VMEM_SHARED((N_SRL, OUT), _f32),
            pltpu.SemaphoreType.DMA,
            pltpu.SemaphoreType.DMA,
        ],
    )
    def k(a_h, b_h, src_h, dst_h, oa, ob, idx_v, r0, r1, tab, g0, g1):
        c = lax.axis_index("c")
        s = lax.axis_index("s")

        def stage(tab_h):
            @pl.when(s < NS - 1)
            def _():
                pltpu.sync_copy(tab_h.at[pl.ds(s * CPA, CPA)],
                                tab.at[pl.ds(s * CPA, CPA)])

            @pl.when(s == NS - 1)
            def _():
                pltpu.sync_copy(tab_h.at[pl.ds((NS - 1) * CPA, CPL)],
                                tab.at[pl.ds((NS - 1) * CPA, CPL)])

        def pipe(idx_h, out_h):
            gbase = s * NCHPT
            pltpu.sync_copy(idx_h.at[pl.ds(gbase, NCHPT)], idx_v)

            def gat(j, buf, sem):
                pltpu.async_copy(tab.at[idx_v.at[j]], buf, sem)

            def drain(buf, sem):
                pltpu.make_async_copy(a_h.at[pl.ds(0, CH)], buf, sem).wait()

            def write(j, buf):
                pltpu.sync_copy(buf, out_h.at[pl.ds((gbase + j) * CH, CH)])

            gat(0, r0, g0)

            def body(k_, carry):
                j0 = 2 * k_
                j1 = j0 + 1
                drain(r0, g0)
                gat(j1, r1, g1)
                write(j0, r0)
                drain(r1, g1)
                gat(jnp.minimum(j0 + 2, NCHPT - 1), r0, g0)
                write(j1, r1)
                return carry

            lax.fori_loop(0, NCHPT // 2, body, 0)
            drain(r0, g0)

        @pl.when(c == 0)
        def _():
            stage(a_h)

        @pl.when(c == 1)
        def _():
            stage(b_h)

        plsc.subcore_barrier()

        @pl.when(c == 0)
        def _():
            pipe(src_h, oa)

        @pl.when(c == 1)
        def _():
            pipe(dst_h, ob)

    return k(a_tab, b_tab, src_g, dst_g)


_BE = 2048


def _edge_body(a_ref, b_ref, st_ref, ln_ref, g_ref,
               w1_ref, b1_ref, w2_ref, b2_ref, aw1_ref, ex_ref, p_ref):
    st = st_ref[...]
    ln = ln_ref[...]
    yy = st + ln + 1
    inv = _f32(1.0) / (ln.astype(_f32) + _f32(1.0))
    tt = lax.broadcasted_iota(jnp.int32, (GP, _BE), 0)
    mt = (jnp.where(tt == yy, inv, _f32(0.0))
          - jnp.where(tt == st, inv, _f32(0.0)))
    rel = lax.dot_general(mt, g_ref[...], (((0,), (0,)), ((), ())),
                          preferred_element_type=_f32)
    pre = a_ref[...] + rel
    u = jnp.dot(pre, w1_ref[...], preferred_element_type=_f32) + b1_ref[...]
    u = _leaky(u)
    m = jnp.dot(u, w2_ref[...], preferred_element_type=_f32) + b2_ref[...]
    e = jnp.dot(m, aw1_ref[...], preferred_element_type=_f32) + b_ref[...]
    e = _leaky(e)
    ex = jnp.exp(e)
    ex_ref[...] = ex
    p_ref[...] = ex * m


def _tc_edge(a_src, b_dst, st_row, ln_row, g_tab, w1, b1, w2, b2, aw1):
    nblk = E_PAD // _BE
    edge_spec = pl.BlockSpec((_BE, OUT), lambda i: (i, 0))
    row_spec = pl.BlockSpec((1, _BE), lambda i: (0, i))
    g_spec = pl.BlockSpec((GP, OUT), lambda i: (0, 0))
    w_spec = pl.BlockSpec((IN, OUT), lambda i: (0, 0))
    bias_spec = pl.BlockSpec((1, OUT), lambda i: (0, 0))
    return pl.pallas_call(
        _edge_body,
        grid=(nblk,),
        in_specs=[edge_spec, edge_spec, row_spec, row_spec, g_spec,
                  w_spec, bias_spec, w_spec, bias_spec, w_spec],
        out_specs=[edge_spec, edge_spec],
        out_shape=(
            jax.ShapeDtypeStruct((E_PAD, OUT), _f32),
            jax.ShapeDtypeStruct((E_PAD, OUT), _f32),
        ),
    )(a_src, b_dst, st_row, ln_row, g_tab, w1, b1, w2, b2, aw1)


def _sc_scatter(ex, p, dst_s, zrows):
    @functools.partial(
        pl.kernel,
        out_type=(
            jax.ShapeDtypeStruct((N_SRL, OUT), _f32),
            jax.ShapeDtypeStruct((N_SRL, OUT), _f32),
        ),
        mesh=_MESH,
        scratch_types=[
            pltpu.VMEM((NCH2, CH), jnp.int32),
            pltpu.VMEM((CH, OUT), _f32),
            pltpu.VMEM((CH, OUT), _f32),
            pltpu.VMEM_SHARED((ACC_N, OUT), _f32),
            pltpu.SemaphoreType.DMA,
            pltpu.SemaphoreType.DMA,
        ],
    )
    def k(ex_h, p_h, dst_h, z_h, den_o, num_o, idx_v, rows0, rows1, acc,
          r0, r1):
        c = lax.axis_index("c")
        s = lax.axis_index("s")
        pltpu.sync_copy(z_h, acc.at[pl.ds(s * ZPT, ZPT)])
        pltpu.sync_copy(dst_h.at[s], idx_v)
        plsc.subcore_barrier()

        def run(src_h):
            def read(j, buf, sem):
                pltpu.async_copy(src_h.at[pl.ds(s * EPT + j * CH, CH)],
                                 buf, sem)

            def drain(buf, sem):
                pltpu.make_async_copy(src_h.at[pl.ds(0, CH)], buf, sem).wait()

            def scat(j, buf):
                pltpu.sync_copy(buf, acc.at[idx_v.at[j]], add=True)

            read(0, rows0, r0)

            def body(k_, carry):
                j0 = 2 * k_
                j1 = j0 + 1
                drain(rows0, r0)
                read(j1, rows1, r1)
                scat(j0, rows0)
                drain(rows1, r1)
                read(jnp.minimum(j0 + 2, NCH2 - 1), rows0, r0)
                scat(j1, rows1)
                return carry

            lax.fori_loop(0, NCH2 // 2, body, 0)
            drain(rows0, r0)

        @pl.when(c == 0)
        def _():
            run(ex_h)

        @pl.when(c == 1)
        def _():
            run(p_h)

        plsc.subcore_barrier()

        @pl.when((c == 0) & (s < NS - 1))
        def _():
            pltpu.sync_copy(acc.at[pl.ds(s * CPA, CPA)],
                            den_o.at[pl.ds(s * CPA, CPA)])

        @pl.when((c == 0) & (s == NS - 1))
        def _():
            pltpu.sync_copy(acc.at[pl.ds((NS - 1) * CPA, CPL)],
                            den_o.at[pl.ds((NS - 1) * CPA, CPL)])

        @pl.when((c == 1) & (s < NS - 1))
        def _():
            pltpu.sync_copy(acc.at[pl.ds(s * CPA, CPA)],
                            num_o.at[pl.ds(s * CPA, CPA)])

        @pl.when((c == 1) & (s == NS - 1))
        def _():
            pltpu.sync_copy(acc.at[pl.ds((NS - 1) * CPA, CPL)],
                            num_o.at[pl.ds((NS - 1) * CPA, CPL)])

    return k(ex, p, dst_s, zrows)


def _fin_body(den_ref, num_ref, fs_ref, out_ref):
    den = den_ref[...]
    keep = den > 0
    safe = jnp.where(keep, den, _f32(1.0))
    out_ref[...] = jnp.where(keep, num_ref[...] / safe, fs_ref[...])


def _tc_fin(denom, numer, fs):
    return pl.pallas_call(
        _fin_body,
        out_shape=jax.ShapeDtypeStruct((N_SRL, IN), _f32),
    )(denom, numer, fs)


def _sc_s2t(h_srl, s2_g, d2_g, zrows):
    @functools.partial(
        pl.kernel,
        out_type=jax.ShapeDtypeStruct((NC, N_TOK, OUT), _f32),
        mesh=_MESH,
        scratch_types=[
            pltpu.VMEM((K0, CH), jnp.int32),
            pltpu.VMEM((K0, CH), jnp.int32),
            pltpu.VMEM((CH, OUT), _f32),
            pltpu.VMEM((CH, OUT), _f32),
            pltpu.VMEM_SHARED((ACC_N, OUT), _f32),
            pltpu.SemaphoreType.DMA,
            pltpu.SemaphoreType.DMA,
        ],
    )
    def k(h_h, s2_h, d2_h, z_h, out_o, s2_v, d2_v, rows0, rows1, acc, g0, g1):
        c = lax.axis_index("c")
        s = lax.axis_index("s")
        pltpu.sync_copy(z_h, acc.at[pl.ds(s * ZPT, ZPT)])
        plsc.subcore_barrier()

        def pipe(kk, cbase):
            gbase = cbase + s * kk
            pltpu.sync_copy(s2_h.at[pl.ds(gbase, kk)], s2_v.at[pl.ds(0, kk)])
            pltpu.sync_copy(d2_h.at[pl.ds(gbase, kk)], d2_v.at[pl.ds(0, kk)])

            def gat(j, buf, sem):
                pltpu.async_copy(h_h.at[s2_v.at[j]], buf, sem)

            def drain(buf, sem):
                pltpu.make_async_copy(h_h.at[s2_v.at[0]], buf, sem).wait()

            def scat(j, buf):
                pltpu.sync_copy(buf, acc.at[d2_v.at[j]], add=True)

            gat(0, rows0, g0)

            def body(k_, carry):
                j0 = 2 * k_
                j1 = j0 + 1
                drain(rows0, g0)
                gat(j1, rows1, g1)
                scat(j0, rows0)
                drain(rows1, g1)
                gat(jnp.minimum(j0 + 2, kk - 1), rows0, g0)
                scat(j1, rows1)
                return carry

            lax.fori_loop(0, kk // 2, body, 0)
            drain(rows0, g0)

        @pl.when(c == 0)
        def _():
            pipe(K0, 0)

        @pl.when(c == 1)
        def _():
            pipe(K1, CB1)

        plsc.subcore_barrier()

        @pl.when(s < NS - 1)
        def _():
            pltpu.sync_copy(acc.at[pl.ds(s * CPA, CPA)],
                            out_o.at[c, pl.ds(s * CPA, CPA)])

        @pl.when(s == NS - 1)
        def _():
            pltpu.sync_copy(acc.at[pl.ds((NS - 1) * CPA, CPL)],
                            out_o.at[c, pl.ds((NS - 1) * CPA, CPL)])

    return k(h_srl, s2_g, d2_g, zrows)


def _gru_body(p0_ref, p1_ref, htok_ref, wih_ref, whh_ref, bih_ref, bhh_ref,
              out_ref):
    x1 = p0_ref[...] + p1_ref[...]
    h = jnp.zeros((N_TOK, OUT), _f32)
    for x in (x1, htok_ref[...]):
        gi = jnp.dot(x, wih_ref[...], preferred_element_type=_f32) + bih_ref[...]
        gh = jnp.dot(h, whh_ref[...], preferred_element_type=_f32) + bhh_ref[...]
        ir, iz, inn = gi[:, :OUT], gi[:, OUT:2 * OUT], gi[:, 2 * OUT:]
        hr, hz, hn = gh[:, :OUT], gh[:, OUT:2 * OUT], gh[:, 2 * OUT:]
        r = jax.nn.sigmoid(ir + hr)
        z = jax.nn.sigmoid(iz + hz)
        n = jnp.tanh(inn + r * hn)
        h = (1.0 - z) * n + z * h
    out_ref[...] = h


def _tc_gru(p0, p1, htok, wih, whh, bih, bhh):
    return pl.pallas_call(
        _gru_body,
        out_shape=jax.ShapeDtypeStruct((N_TOK, OUT), _f32),
    )(p0, p1, htok, wih, whh, bih, bhh)


def kernel(feat_srl, feat_tok, bert_token_emb, edge_index_rel, span_start,
           span_len, edge_index_s2t, rel_W, rel_b, nt_W1, nt_b1, nt_W2, nt_b2,
           att_W, att_b, gru_Wih, gru_Whh, gru_bih, gru_bhh):
    rwa = rel_W[:IN]
    rwb = rel_W[IN:]
    aw1 = att_W[:OUT]
    aw2 = att_W[OUT:]
    rb = rel_b.reshape(1, IN)
    b1 = nt_b1.reshape(1, IN)
    b2 = nt_b2.reshape(1, OUT)
    ab = att_b.reshape(1, OUT)
    bih = gru_bih.reshape(1, 3 * OUT)
    bhh = gru_bhh.reshape(1, 3 * OUT)

    a_tab, g_tab, b_tab, h_tok = _tc_prep(
        bert_token_emb, feat_srl, feat_tok, rwa, rwb, rb,
        nt_W1, b1, nt_W2, b2, aw2, ab)

    npad = E_PAD - E_REL
    zpad = jnp.zeros((npad,), jnp.int32)
    trash = jnp.full((npad,), N_SRL, jnp.int32)
    src = jnp.concatenate([edge_index_rel[0].astype(jnp.int32), zpad])
    dst = jnp.concatenate([edge_index_rel[1].astype(jnp.int32), trash])
    src_g = src.reshape(NCHT, CH)
    dst_g = jnp.where(dst >= N_SRL, 0, dst).reshape(NCHT, CH)
    st_row = jnp.concatenate(
        [span_start.astype(jnp.int32), zpad]).reshape(1, E_PAD)
    ln_row = jnp.concatenate(
        [span_len.astype(jnp.int32), zpad]).reshape(1, E_PAD)

    a_src, b_dst = _sc_gather(a_tab, b_tab, src_g, dst_g)

    ex, p = _tc_edge(a_src, b_dst, st_row, ln_row, g_tab,
                     nt_W1, b1, nt_W2, b2, aw1)

    zrows = jnp.zeros((ZPT, OUT), _f32)
    dst_s = dst.reshape(NS, NCH2, CH)
    denom, numer = _sc_scatter(ex, p, dst_s, zrows)

    h_srl = _tc_fin(denom, numer, feat_srl)

    s2_g = jnp.concatenate(
        [edge_index_s2t[0].astype(jnp.int32), zpad]).reshape(NCHT, CH)
    d2_g = jnp.concatenate(
        [edge_index_s2t[1].astype(jnp.int32), trash]).reshape(NCHT, CH)
    partials = _sc_s2t(h_srl, s2_g, d2_g, zrows)

    h_out = _tc_gru(partials[0], partials[1], h_tok, gru_Wih, gru_Whh, bih, bhh)
    return (h_srl, h_out)


# speedup vs baseline: 7.9430x; 1.0386x over previous
import functools

import jax
import jax.numpy as jnp
from jax import lax
from jax.experimental import pallas as pl
from jax.experimental.pallas import tpu as pltpu
from jax.experimental.pallas import tpu_sc as plsc

IN = 128
OUT = 128
BERT = 768
N_SRL = 10000
N_TOK = 10000
E_REL = 160000
E_S2T = 160000
T = 512
NSTART = 500
NLEN = 8
NSPAN = NLEN * NSTART

NC = 2
NS = 16
NW = NC * NS
CH = 128
E_PAD = 163840
EPW = E_PAD // NW
NCH = EPW // CH
EPT = E_PAD // NS
NCH2 = EPT // CH
ACC_N = 10240
ZPT = ACC_N // NS
CPA = 632
CPL = N_SRL - 15 * CPA

NCHT = E_PAD // CH
K0 = 56
K1 = (NCHT // NS) - K0
CB1 = NS * K0
NCHPT = NCHT // NS
GP = 512

_f32 = jnp.float32


def _leaky(x):
    return jnp.where(x >= 0, x, 0.01 * x)


def _prep_body(bert_ref, fs_ref, ft_ref, rwa_ref, rwb_ref, rb_ref,
               w1_ref, b1_ref, w2_ref, b2_ref, aw2_ref, ab_ref,
               a_ref, r_ref, batt_ref, htok_ref):
    r_ref[...] = jnp.dot(bert_ref[...], rwb_ref[...],
                         preferred_element_type=_f32)

    def ntrans(x):
        u = jnp.dot(x, w1_ref[...], preferred_element_type=_f32) + b1_ref[...]
        u = _leaky(u)
        return jnp.dot(u, w2_ref[...], preferred_element_type=_f32) + b2_ref[...]

    fs = fs_ref[...]
    a_ref[...] = jnp.dot(fs, rwa_ref[...], preferred_element_type=_f32) + rb_ref[...]
    t = ntrans(fs)
    batt_ref[...] = jnp.dot(t, aw2_ref[...], preferred_element_type=_f32) + ab_ref[...]
    htok_ref[...] = ntrans(ft_ref[...])


def _tc_prep(bert, fs, ft, rwa, rwb, rb, w1, b1, w2, b2, aw2, ab):
    return pl.pallas_call(
        _prep_body,
        out_shape=(
            jax.ShapeDtypeStruct((N_SRL, IN), _f32),
            jax.ShapeDtypeStruct((GP, OUT), _f32),
            jax.ShapeDtypeStruct((N_SRL, OUT), _f32),
            jax.ShapeDtypeStruct((N_TOK, OUT), _f32),
        ),
    )(bert, fs, ft, rwa, rwb, rb, w1, b1, w2, b2, aw2, ab)


_MESH = plsc.VectorSubcoreMesh(core_axis_name="c", subcore_axis_name="s",
                               num_cores=NC, num_subcores=NS)


def _sc_gather(a_tab, b_tab, src_g, dst_g):
    @functools.partial(
        pl.kernel,
        out_type=(
            jax.ShapeDtypeStruct((E_PAD, OUT), _f32),
            jax.ShapeDtypeStruct((E_PAD, OUT), _f32),
        ),
        mesh=_MESH,
        scratch_types=[
            pltpu.VMEM((NCHPT, CH), jnp.int32),
            pltpu.VMEM((CH, OUT), _f32),
            pltpu.VMEM((CH, OUT), _f32),
            pltpu.VMEM_SHARED((N_SRL, OUT), _f32),
            pltpu.SemaphoreType.DMA,
            pltpu.SemaphoreType.DMA,
        ],
    )
    def k(a_h, b_h, src_h, dst_h, oa, ob, idx_v, r0, r1, tab, g0, g1):
        c = lax.axis_index("c")
        s = lax.axis_index("s")

        def stage(tab_h):
            @pl.when(s < NS - 1)
            def _():
                pltpu.sync_copy(tab_h.at[pl.ds(s * CPA, CPA)],
                                tab.at[pl.ds(s * CPA, CPA)])

            @pl.when(s == NS - 1)
            def _():
                pltpu.sync_copy(tab_h.at[pl.ds((NS - 1) * CPA, CPL)],
                                tab.at[pl.ds((NS - 1) * CPA, CPL)])

        def pipe(idx_h, out_h):
            gbase = s * NCHPT
            pltpu.sync_copy(idx_h.at[pl.ds(gbase, NCHPT)], idx_v)

            def gat(j, buf, sem):
                pltpu.async_copy(tab.at[idx_v.at[j]], buf, sem)

            def drain(buf, sem):
                pltpu.make_async_copy(a_h.at[pl.ds(0, CH)], buf, sem).wait()

            def write(j, buf):
                pltpu.sync_copy(buf, out_h.at[pl.ds((gbase + j) * CH, CH)])

            gat(0, r0, g0)

            def body(k_, carry):
                j0 = 2 * k_
                j1 = j0 + 1
                drain(r0, g0)
                gat(j1, r1, g1)
                write(j0, r0)
                drain(r1, g1)
                gat(jnp.minimum(j0 + 2, NCHPT - 1), r0, g0)
                write(j1, r1)
                return carry

            lax.fori_loop(0, NCHPT // 2, body, 0)
            drain(r0, g0)

        @pl.when(c == 0)
        def _():
            stage(a_h)

        @pl.when(c == 1)
        def _():
            stage(b_h)

        plsc.subcore_barrier()

        @pl.when(c == 0)
        def _():
            pipe(src_h, oa)

        @pl.when(c == 1)
        def _():
            pipe(dst_h, ob)

    return k(a_tab, b_tab, src_g, dst_g)


_BE = 2048


def _edge_body(a_ref, b_ref, st_ref, ln_ref, g_ref,
               w1_ref, b1_ref, w2_ref, b2_ref, aw1_ref, ex_ref, p_ref):
    st = st_ref[...]
    ln = ln_ref[...]
    yy = st + ln + 1
    inv = _f32(1.0) / (ln.astype(_f32) + _f32(1.0))
    tt = lax.broadcasted_iota(jnp.int32, (GP, _BE), 0)
    mt = jnp.where((tt >= st) & (tt < yy), inv, _f32(0.0))
    rel = lax.dot_general(mt, g_ref[...], (((0,), (0,)), ((), ())),
                          preferred_element_type=_f32)
    pre = a_ref[...] + rel
    u = jnp.dot(pre, w1_ref[...], preferred_element_type=_f32) + b1_ref[...]
    u = _leaky(u)
    m = jnp.dot(u, w2_ref[...], preferred_element_type=_f32) + b2_ref[...]
    e = jnp.dot(m, aw1_ref[...], preferred_element_type=_f32) + b_ref[...]
    e = _leaky(e)
    ex = jnp.exp(e)
    ex_ref[...] = ex
    p_ref[...] = ex * m


def _tc_edge(a_src, b_dst, st_row, ln_row, g_tab, w1, b1, w2, b2, aw1):
    nblk = E_PAD // _BE
    edge_spec = pl.BlockSpec((_BE, OUT), lambda i: (i, 0))
    row_spec = pl.BlockSpec((1, _BE), lambda i: (0, i))
    g_spec = pl.BlockSpec((GP, OUT), lambda i: (0, 0))
    w_spec = pl.BlockSpec((IN, OUT), lambda i: (0, 0))
    bias_spec = pl.BlockSpec((1, OUT), lambda i: (0, 0))
    return pl.pallas_call(
        _edge_body,
        grid=(nblk,),
        in_specs=[edge_spec, edge_spec, row_spec, row_spec, g_spec,
                  w_spec, bias_spec, w_spec, bias_spec, w_spec],
        out_specs=[edge_spec, edge_spec],
        out_shape=(
            jax.ShapeDtypeStruct((E_PAD, OUT), _f32),
            jax.ShapeDtypeStruct((E_PAD, OUT), _f32),
        ),
    )(a_src, b_dst, st_row, ln_row, g_tab, w1, b1, w2, b2, aw1)


def _sc_scatter(ex, p, dst_s, zrows):
    @functools.partial(
        pl.kernel,
        out_type=(
            jax.ShapeDtypeStruct((N_SRL, OUT), _f32),
            jax.ShapeDtypeStruct((N_SRL, OUT), _f32),
        ),
        mesh=_MESH,
        scratch_types=[
            pltpu.VMEM((NCH2, CH), jnp.int32),
            pltpu.VMEM((CH, OUT), _f32),
            pltpu.VMEM((CH, OUT), _f32),
            pltpu.VMEM_SHARED((ACC_N, OUT), _f32),
            pltpu.SemaphoreType.DMA,
            pltpu.SemaphoreType.DMA,
        ],
    )
    def k(ex_h, p_h, dst_h, z_h, den_o, num_o, idx_v, rows0, rows1, acc,
          r0, r1):
        c = lax.axis_index("c")
        s = lax.axis_index("s")
        pltpu.sync_copy(z_h, acc.at[pl.ds(s * ZPT, ZPT)])
        pltpu.sync_copy(dst_h.at[s], idx_v)
        plsc.subcore_barrier()

        def run(src_h):
            def read(j, buf, sem):
                pltpu.async_copy(src_h.at[pl.ds(s * EPT + j * CH, CH)],
                                 buf, sem)

            def drain(buf, sem):
                pltpu.make_async_copy(src_h.at[pl.ds(0, CH)], buf, sem).wait()

            def scat(j, buf):
                pltpu.sync_copy(buf, acc.at[idx_v.at[j]], add=True)

            read(0, rows0, r0)

            def body(k_, carry):
                j0 = 2 * k_
                j1 = j0 + 1
                drain(rows0, r0)
                read(j1, rows1, r1)
                scat(j0, rows0)
                drain(rows1, r1)
                read(jnp.minimum(j0 + 2, NCH2 - 1), rows0, r0)
                scat(j1, rows1)
                return carry

            lax.fori_loop(0, NCH2 // 2, body, 0)
            drain(rows0, r0)

        @pl.when(c == 0)
        def _():
            run(ex_h)

        @pl.when(c == 1)
        def _():
            run(p_h)

        plsc.subcore_barrier()

        @pl.when((c == 0) & (s < NS - 1))
        def _():
            pltpu.sync_copy(acc.at[pl.ds(s * CPA, CPA)],
                            den_o.at[pl.ds(s * CPA, CPA)])

        @pl.when((c == 0) & (s == NS - 1))
        def _():
            pltpu.sync_copy(acc.at[pl.ds((NS - 1) * CPA, CPL)],
                            den_o.at[pl.ds((NS - 1) * CPA, CPL)])

        @pl.when((c == 1) & (s < NS - 1))
        def _():
            pltpu.sync_copy(acc.at[pl.ds(s * CPA, CPA)],
                            num_o.at[pl.ds(s * CPA, CPA)])

        @pl.when((c == 1) & (s == NS - 1))
        def _():
            pltpu.sync_copy(acc.at[pl.ds((NS - 1) * CPA, CPL)],
                            num_o.at[pl.ds((NS - 1) * CPA, CPL)])

    return k(ex, p, dst_s, zrows)


def _fin_body(den_ref, num_ref, fs_ref, out_ref):
    den = den_ref[...]
    keep = den > 0
    safe = jnp.where(keep, den, _f32(1.0))
    out_ref[...] = jnp.where(keep, num_ref[...] / safe, fs_ref[...])


def _tc_fin(denom, numer, fs):
    return pl.pallas_call(
        _fin_body,
        out_shape=jax.ShapeDtypeStruct((N_SRL, IN), _f32),
    )(denom, numer, fs)


def _sc_s2t(h_srl, s2_g, d2_g, zrows):
    @functools.partial(
        pl.kernel,
        out_type=jax.ShapeDtypeStruct((NC, N_TOK, OUT), _f32),
        mesh=_MESH,
        scratch_types=[
            pltpu.VMEM((K0, CH), jnp.int32),
            pltpu.VMEM((K0, CH), jnp.int32),
            pltpu.VMEM((CH, OUT), _f32),
            pltpu.VMEM((CH, OUT), _f32),
            pltpu.VMEM_SHARED((ACC_N, OUT), _f32),
            pltpu.SemaphoreType.DMA,
            pltpu.SemaphoreType.DMA,
        ],
    )
    def k(h_h, s2_h, d2_h, z_h, out_o, s2_v, d2_v, rows0, rows1, acc, g0, g1):
        c = lax.axis_index("c")
        s = lax.axis_index("s")
        pltpu.sync_copy(z_h, acc.at[pl.ds(s * ZPT, ZPT)])
        plsc.subcore_barrier()

        def pipe(kk, cbase):
            gbase = cbase + s * kk
            pltpu.sync_copy(s2_h.at[pl.ds(gbase, kk)], s2_v.at[pl.ds(0, kk)])
            pltpu.sync_copy(d2_h.at[pl.ds(gbase, kk)], d2_v.at[pl.ds(0, kk)])

            def gat(j, buf, sem):
                pltpu.async_copy(h_h.at[s2_v.at[j]], buf, sem)

            def drain(buf, sem):
                pltpu.make_async_copy(h_h.at[s2_v.at[0]], buf, sem).wait()

            def scat(j, buf):
                pltpu.sync_copy(buf, acc.at[d2_v.at[j]], add=True)

            gat(0, rows0, g0)

            def body(k_, carry):
                j0 = 2 * k_
                j1 = j0 + 1
                drain(rows0, g0)
                gat(j1, rows1, g1)
                scat(j0, rows0)
                drain(rows1, g1)
                gat(jnp.minimum(j0 + 2, kk - 1), rows0, g0)
                scat(j1, rows1)
                return carry

            lax.fori_loop(0, kk // 2, body, 0)
            drain(rows0, g0)

        @pl.when(c == 0)
        def _():
            pipe(K0, 0)

        @pl.when(c == 1)
        def _():
            pipe(K1, CB1)

        plsc.subcore_barrier()

        @pl.when(s < NS - 1)
        def _():
            pltpu.sync_copy(acc.at[pl.ds(s * CPA, CPA)],
                            out_o.at[c, pl.ds(s * CPA, CPA)])

        @pl.when(s == NS - 1)
        def _():
            pltpu.sync_copy(acc.at[pl.ds((NS - 1) * CPA, CPL)],
                            out_o.at[c, pl.ds((NS - 1) * CPA, CPL)])

    return k(h_srl, s2_g, d2_g, zrows)


def _gru_body(p0_ref, p1_ref, htok_ref, wih_ref, whh_ref, bih_ref, bhh_ref,
              out_ref):
    x1 = p0_ref[...] + p1_ref[...]
    h = jnp.zeros((N_TOK, OUT), _f32)
    for x in (x1, htok_ref[...]):
        gi = jnp.dot(x, wih_ref[...], preferred_element_type=_f32) + bih_ref[...]
        gh = jnp.dot(h, whh_ref[...], preferred_element_type=_f32) + bhh_ref[...]
        ir, iz, inn = gi[:, :OUT], gi[:, OUT:2 * OUT], gi[:, 2 * OUT:]
        hr, hz, hn = gh[:, :OUT], gh[:, OUT:2 * OUT], gh[:, 2 * OUT:]
        r = jax.nn.sigmoid(ir + hr)
        z = jax.nn.sigmoid(iz + hz)
        n = jnp.tanh(inn + r * hn)
        h = (1.0 - z) * n + z * h
    out_ref[...] = h


def _tc_gru(p0, p1, htok, wih, whh, bih, bhh):
    return pl.pallas_call(
        _gru_body,
        out_shape=jax.ShapeDtypeStruct((N_TOK, OUT), _f32),
    )(p0, p1, htok, wih, whh, bih, bhh)


def kernel(feat_srl, feat_tok, bert_token_emb, edge_index_rel, span_start,
           span_len, edge_index_s2t, rel_W, rel_b, nt_W1, nt_b1, nt_W2, nt_b2,
           att_W, att_b, gru_Wih, gru_Whh, gru_bih, gru_bhh):
    rwa = rel_W[:IN]
    rwb = rel_W[IN:]
    aw1 = att_W[:OUT]
    aw2 = att_W[OUT:]
    rb = rel_b.reshape(1, IN)
    b1 = nt_b1.reshape(1, IN)
    b2 = nt_b2.reshape(1, OUT)
    ab = att_b.reshape(1, OUT)
    bih = gru_bih.reshape(1, 3 * OUT)
    bhh = gru_bhh.reshape(1, 3 * OUT)

    a_tab, g_tab, b_tab, h_tok = _tc_prep(
        bert_token_emb, feat_srl, feat_tok, rwa, rwb, rb,
        nt_W1, b1, nt_W2, b2, aw2, ab)

    npad = E_PAD - E_REL
    zpad = jnp.zeros((npad,), jnp.int32)
    trash = jnp.full((npad,), N_SRL, jnp.int32)
    src = jnp.concatenate([edge_index_rel[0].astype(jnp.int32), zpad])
    dst = jnp.concatenate([edge_index_rel[1].astype(jnp.int32), trash])
    src_g = src.reshape(NCHT, CH)
    dst_g = jnp.where(dst >= N_SRL, 0, dst).reshape(NCHT, CH)
    st_row = jnp.concatenate(
        [span_start.astype(jnp.int32), zpad]).reshape(1, E_PAD)
    ln_row = jnp.concatenate(
        [span_len.astype(jnp.int32), zpad]).reshape(1, E_PAD)

    a_src, b_dst = _sc_gather(a_tab, b_tab, src_g, dst_g)

    ex, p = _tc_edge(a_src, b_dst, st_row, ln_row, g_tab,
                     nt_W1, b1, nt_W2, b2, aw1)

    zrows = jnp.zeros((ZPT, OUT), _f32)
    dst_s = dst.reshape(NS, NCH2, CH)
    denom, numer = _sc_scatter(ex, p, dst_s, zrows)

    h_srl = _tc_fin(denom, numer, feat_srl)

    s2_g = jnp.concatenate(
        [edge_index_s2t[0].astype(jnp.int32), zpad]).reshape(NCHT, CH)
    d2_g = jnp.concatenate(
        [edge_index_s2t[1].astype(jnp.int32), trash]).reshape(NCHT, CH)
    partials = _sc_s2t(h_srl, s2_g, d2_g, zrows)

    h_out = _tc_gru(partials[0], partials[1], h_tok, gru_Wih, gru_Whh, bih, bhh)
    return (h_srl, h_out)


# speedup vs baseline: 7.9446x; 1.0002x over previous
import functools

import jax
import jax.numpy as jnp
from jax import lax
from jax.experimental import pallas as pl
from jax.experimental.pallas import tpu as pltpu
from jax.experimental.pallas import tpu_sc as plsc

IN = 128
OUT = 128
BERT = 768
N_SRL = 10000
N_TOK = 10000
E_REL = 160000
E_S2T = 160000
T = 512
NSTART = 500
NLEN = 8
NSPAN = NLEN * NSTART

NC = 2
NS = 16
NW = NC * NS
CH = 128
E_PAD = 163840
EPW = E_PAD // NW
NCH = EPW // CH
EPT = E_PAD // NS
NCH2 = EPT // CH
ACC_N = 10240
ZPT = ACC_N // NS
CPA = 632
CPL = N_SRL - 15 * CPA

NCHT = E_PAD // CH
K0 = 56
K1 = (NCHT // NS) - K0
CB1 = NS * K0
NCHPT = NCHT // NS
GP = 512
HHALF = N_TOK // NC
HZERO = HHALF
HTAB = HHALF + 8
HSA = 312
HSL = HHALF - 15 * HSA

_f32 = jnp.float32


def _leaky(x):
    return jnp.where(x >= 0, x, 0.01 * x)


def _prep_body(bert_ref, fs_ref, ft_ref, rwa_ref, rwb_ref, rb_ref,
               w1_ref, b1_ref, w2_ref, b2_ref, aw2_ref, ab_ref,
               a_ref, r_ref, batt_ref, htok_ref):
    r_ref[...] = jnp.dot(bert_ref[...], rwb_ref[...],
                         preferred_element_type=_f32)

    def ntrans(x):
        u = jnp.dot(x, w1_ref[...], preferred_element_type=_f32) + b1_ref[...]
        u = _leaky(u)
        return jnp.dot(u, w2_ref[...], preferred_element_type=_f32) + b2_ref[...]

    fs = fs_ref[...]
    a_ref[...] = jnp.dot(fs, rwa_ref[...], preferred_element_type=_f32) + rb_ref[...]
    t = ntrans(fs)
    batt_ref[...] = jnp.dot(t, aw2_ref[...], preferred_element_type=_f32) + ab_ref[...]
    htok_ref[...] = ntrans(ft_ref[...])


def _tc_prep(bert, fs, ft, rwa, rwb, rb, w1, b1, w2, b2, aw2, ab):
    return pl.pallas_call(
        _prep_body,
        out_shape=(
            jax.ShapeDtypeStruct((N_SRL, IN), _f32),
            jax.ShapeDtypeStruct((GP, OUT), _f32),
            jax.ShapeDtypeStruct((N_SRL, OUT), _f32),
            jax.ShapeDtypeStruct((N_TOK, OUT), _f32),
        ),
    )(bert, fs, ft, rwa, rwb, rb, w1, b1, w2, b2, aw2, ab)


_MESH = plsc.VectorSubcoreMesh(core_axis_name="c", subcore_axis_name="s",
                               num_cores=NC, num_subcores=NS)


def _sc_gather(a_tab, b_tab, src_g, dst_g):
    @functools.partial(
        pl.kernel,
        out_type=(
            jax.ShapeDtypeStruct((E_PAD, OUT), _f32),
            jax.ShapeDtypeStruct((E_PAD, OUT), _f32),
        ),
        mesh=_MESH,
        scratch_types=[
            pltpu.VMEM((NCHPT, CH), jnp.int32),
            pltpu.VMEM((CH, OUT), _f32),
            pltpu.VMEM((CH, OUT), _f32),
            pltpu.VMEM_SHARED((N_SRL, OUT), _f32),
            pltpu.SemaphoreType.DMA,
            pltpu.SemaphoreType.DMA,
        ],
    )
    def k(a_h, b_h, src_h, dst_h, oa, ob, idx_v, r0, r1, tab, g0, g1):
        c = lax.axis_index("c")
        s = lax.axis_index("s")

        def stage(tab_h):
            @pl.when(s < NS - 1)
            def _():
                pltpu.sync_copy(tab_h.at[pl.ds(s * CPA, CPA)],
                                tab.at[pl.ds(s * CPA, CPA)])

            @pl.when(s == NS - 1)
            def _():
                pltpu.sync_copy(tab_h.at[pl.ds((NS - 1) * CPA, CPL)],
                                tab.at[pl.ds((NS - 1) * CPA, CPL)])

        def pipe(idx_h, out_h):
            gbase = s * NCHPT
            pltpu.sync_copy(idx_h.at[pl.ds(gbase, NCHPT)], idx_v)

            def gat(j, buf, sem):
                pltpu.async_copy(tab.at[idx_v.at[j]], buf, sem)

            def drain(buf, sem):
                pltpu.make_async_copy(a_h.at[pl.ds(0, CH)], buf, sem).wait()

            def write(j, buf):
                pltpu.sync_copy(buf, out_h.at[pl.ds((gbase + j) * CH, CH)])

            gat(0, r0, g0)

            def body(k_, carry):
                j0 = 2 * k_
                j1 = j0 + 1
                drain(r0, g0)
                gat(j1, r1, g1)
                write(j0, r0)
                drain(r1, g1)
                gat(jnp.minimum(j0 + 2, NCHPT - 1), r0, g0)
                write(j1, r1)
                return carry

            lax.fori_loop(0, NCHPT // 2, body, 0)
            drain(r0, g0)

        @pl.when(c == 0)
        def _():
            stage(a_h)

        @pl.when(c == 1)
        def _():
            stage(b_h)

        plsc.subcore_barrier()

        @pl.when(c == 0)
        def _():
            pipe(src_h, oa)

        @pl.when(c == 1)
        def _():
            pipe(dst_h, ob)

    return k(a_tab, b_tab, src_g, dst_g)


_BE = 2048


def _edge_body(a_ref, b_ref, st_ref, ln_ref, g_ref,
               w1_ref, b1_ref, w2_ref, b2_ref, aw1_ref, ex_ref, p_ref):
    st = st_ref[...]
    ln = ln_ref[...]
    yy = st + ln + 1
    inv = _f32(1.0) / (ln.astype(_f32) + _f32(1.0))
    tt = lax.broadcasted_iota(jnp.int32, (GP, _BE), 0)
    mt = jnp.where((tt >= st) & (tt < yy), inv, _f32(0.0))
    rel = lax.dot_general(mt, g_ref[...], (((0,), (0,)), ((), ())),
                          preferred_element_type=_f32)
    pre = a_ref[...] + rel
    u = jnp.dot(pre, w1_ref[...], preferred_element_type=_f32) + b1_ref[...]
    u = _leaky(u)
    m = jnp.dot(u, w2_ref[...], preferred_element_type=_f32) + b2_ref[...]
    e = jnp.dot(m, aw1_ref[...], preferred_element_type=_f32) + b_ref[...]
    e = _leaky(e)
    ex = jnp.exp(e)
    ex_ref[...] = ex
    p_ref[...] = ex * m


def _tc_edge(a_src, b_dst, st_row, ln_row, g_tab, w1, b1, w2, b2, aw1):
    nblk = E_PAD // _BE
    edge_spec = pl.BlockSpec((_BE, OUT), lambda i: (i, 0))
    row_spec = pl.BlockSpec((1, _BE), lambda i: (0, i))
    g_spec = pl.BlockSpec((GP, OUT), lambda i: (0, 0))
    w_spec = pl.BlockSpec((IN, OUT), lambda i: (0, 0))
    bias_spec = pl.BlockSpec((1, OUT), lambda i: (0, 0))
    return pl.pallas_call(
        _edge_body,
        grid=(nblk,),
        in_specs=[edge_spec, edge_spec, row_spec, row_spec, g_spec,
                  w_spec, bias_spec, w_spec, bias_spec, w_spec],
        out_specs=[edge_spec, edge_spec],
        out_shape=(
            jax.ShapeDtypeStruct((E_PAD, OUT), _f32),
            jax.ShapeDtypeStruct((E_PAD, OUT), _f32),
        ),
    )(a_src, b_dst, st_row, ln_row, g_tab, w1, b1, w2, b2, aw1)


def _sc_scatter(ex, p, dst_s, zrows):
    @functools.partial(
        pl.kernel,
        out_type=(
            jax.ShapeDtypeStruct((N_SRL, OUT), _f32),
            jax.ShapeDtypeStruct((N_SRL, OUT), _f32),
        ),
        mesh=_MESH,
        scratch_types=[
            pltpu.VMEM((NCH2, CH), jnp.int32),
            pltpu.VMEM((CH, OUT), _f32),
            pltpu.VMEM((CH, OUT), _f32),
            pltpu.VMEM_SHARED((ACC_N, OUT), _f32),
            pltpu.SemaphoreType.DMA,
            pltpu.SemaphoreType.DMA,
        ],
    )
    def k(ex_h, p_h, dst_h, z_h, den_o, num_o, idx_v, rows0, rows1, acc,
          r0, r1):
        c = lax.axis_index("c")
        s = lax.axis_index("s")
        pltpu.sync_copy(z_h, acc.at[pl.ds(s * ZPT, ZPT)])
        pltpu.sync_copy(dst_h.at[s], idx_v)
        plsc.subcore_barrier()

        def run(src_h):
            def read(j, buf, sem):
                pltpu.async_copy(src_h.at[pl.ds(s * EPT + j * CH, CH)],
                                 buf, sem)

            def drain(buf, sem):
                pltpu.make_async_copy(src_h.at[pl.ds(0, CH)], buf, sem).wait()

            def scat(j, buf):
                pltpu.sync_copy(buf, acc.at[idx_v.at[j]], add=True)

            read(0, rows0, r0)

            def body(k_, carry):
                j0 = 2 * k_
                j1 = j0 + 1
                drain(rows0, r0)
                read(j1, rows1, r1)
                scat(j0, rows0)
                drain(rows1, r1)
                read(jnp.minimum(j0 + 2, NCH2 - 1), rows0, r0)
                scat(j1, rows1)
                return carry

            lax.fori_loop(0, NCH2 // 2, body, 0)
            drain(rows0, r0)

        @pl.when(c == 0)
        def _():
            run(ex_h)

        @pl.when(c == 1)
        def _():
            run(p_h)

        plsc.subcore_barrier()

        @pl.when((c == 0) & (s < NS - 1))
        def _():
            pltpu.sync_copy(acc.at[pl.ds(s * CPA, CPA)],
                            den_o.at[pl.ds(s * CPA, CPA)])

        @pl.when((c == 0) & (s == NS - 1))
        def _():
            pltpu.sync_copy(acc.at[pl.ds((NS - 1) * CPA, CPL)],
                            den_o.at[pl.ds((NS - 1) * CPA, CPL)])

        @pl.when((c == 1) & (s < NS - 1))
        def _():
            pltpu.sync_copy(acc.at[pl.ds(s * CPA, CPA)],
                            num_o.at[pl.ds(s * CPA, CPA)])

        @pl.when((c == 1) & (s == NS - 1))
        def _():
            pltpu.sync_copy(acc.at[pl.ds((NS - 1) * CPA, CPL)],
                            num_o.at[pl.ds((NS - 1) * CPA, CPL)])

    return k(ex, p, dst_s, zrows)


def _fin_body(den_ref, num_ref, fs_ref, out_ref):
    den = den_ref[...]
    keep = den > 0
    safe = jnp.where(keep, den, _f32(1.0))
    out_ref[...] = jnp.where(keep, num_ref[...] / safe, fs_ref[...])


def _tc_fin(denom, numer, fs):
    return pl.pallas_call(
        _fin_body,
        out_shape=jax.ShapeDtypeStruct((N_SRL, IN), _f32),
    )(denom, numer, fs)


def _sc_s2t(h_srl, s2_g, d2_g, zrows):
    @functools.partial(
        pl.kernel,
        out_type=jax.ShapeDtypeStruct((NC, N_TOK, OUT), _f32),
        mesh=_MESH,
        scratch_types=[
            pltpu.VMEM((NCH, CH), jnp.int32),
            pltpu.VMEM((NCH, CH), jnp.int32),
            pltpu.VMEM((CH, OUT), _f32),
            pltpu.VMEM((CH, OUT), _f32),
            pltpu.VMEM_SHARED((ACC_N, OUT), _f32),
            pltpu.SemaphoreType.DMA,
            pltpu.SemaphoreType.DMA,
        ],
    )
    def k(h_h, s2_h, d2_h, z_h, out_o, s2_v, d2_v, rows0, rows1, acc, g0, g1):
        c = lax.axis_index("c")
        s = lax.axis_index("s")
        wid = s * NC + c
        pltpu.sync_copy(z_h, acc.at[pl.ds(s * ZPT, ZPT)])
        gbase = wid * NCH
        pltpu.sync_copy(s2_h.at[pl.ds(gbase, NCH)], s2_v)
        pltpu.sync_copy(d2_h.at[pl.ds(gbase, NCH)], d2_v)
        plsc.subcore_barrier()

        def gat(j, buf, sem):
            pltpu.async_copy(h_h.at[s2_v.at[j]], buf, sem)

        def drain(buf, sem):
            pltpu.make_async_copy(h_h.at[pl.ds(0, CH)], buf, sem).wait()

        def scat(j, buf):
            pltpu.sync_copy(buf, acc.at[d2_v.at[j]], add=True)

        gat(0, rows0, g0)

        def body(k_, carry):
            j0 = 2 * k_
            j1 = j0 + 1
            drain(rows0, g0)
            gat(j1, rows1, g1)
            scat(j0, rows0)
            drain(rows1, g1)
            gat(jnp.minimum(j0 + 2, NCH - 1), rows0, g0)
            scat(j1, rows1)
            return carry

        lax.fori_loop(0, NCH // 2, body, 0)
        drain(rows0, g0)
        plsc.subcore_barrier()

        @pl.when(s < NS - 1)
        def _():
            pltpu.sync_copy(acc.at[pl.ds(s * CPA, CPA)],
                            out_o.at[c, pl.ds(s * CPA, CPA)])

        @pl.when(s == NS - 1)
        def _():
            pltpu.sync_copy(acc.at[pl.ds((NS - 1) * CPA, CPL)],
                            out_o.at[c, pl.ds((NS - 1) * CPA, CPL)])

    return k(h_srl, s2_g, d2_g, zrows)


def _gru_body(p0_ref, p1_ref, htok_ref, wih_ref, whh_ref, bih_ref, bhh_ref,
              out_ref):
    x1 = p0_ref[...] + p1_ref[...]
    h = jnp.zeros((N_TOK, OUT), _f32)
    for x in (x1, htok_ref[...]):
        gi = jnp.dot(x, wih_ref[...], preferred_element_type=_f32) + bih_ref[...]
        gh = jnp.dot(h, whh_ref[...], preferred_element_type=_f32) + bhh_ref[...]
        ir, iz, inn = gi[:, :OUT], gi[:, OUT:2 * OUT], gi[:, 2 * OUT:]
        hr, hz, hn = gh[:, :OUT], gh[:, OUT:2 * OUT], gh[:, 2 * OUT:]
        r = jax.nn.sigmoid(ir + hr)
        z = jax.nn.sigmoid(iz + hz)
        n = jnp.tanh(inn + r * hn)
        h = (1.0 - z) * n + z * h
    out_ref[...] = h


def _tc_gru(p0, p1, htok, wih, whh, bih, bhh):
    return pl.pallas_call(
        _gru_body,
        out_shape=jax.ShapeDtypeStruct((N_TOK, OUT), _f32),
    )(p0, p1, htok, wih, whh, bih, bhh)


def kernel(feat_srl, feat_tok, bert_token_emb, edge_index_rel, span_start,
           span_len, edge_index_s2t, rel_W, rel_b, nt_W1, nt_b1, nt_W2, nt_b2,
           att_W, att_b, gru_Wih, gru_Whh, gru_bih, gru_bhh):
    rwa = rel_W[:IN]
    rwb = rel_W[IN:]
    aw1 = att_W[:OUT]
    aw2 = att_W[OUT:]
    rb = rel_b.reshape(1, IN)
    b1 = nt_b1.reshape(1, IN)
    b2 = nt_b2.reshape(1, OUT)
    ab = att_b.reshape(1, OUT)
    bih = gru_bih.reshape(1, 3 * OUT)
    bhh = gru_bhh.reshape(1, 3 * OUT)

    a_tab, g_tab, b_tab, h_tok = _tc_prep(
        bert_token_emb, feat_srl, feat_tok, rwa, rwb, rb,
        nt_W1, b1, nt_W2, b2, aw2, ab)

    npad = E_PAD - E_REL
    zpad = jnp.zeros((npad,), jnp.int32)
    trash = jnp.full((npad,), N_SRL, jnp.int32)
    src = jnp.concatenate([edge_index_rel[0].astype(jnp.int32), zpad])
    dst = jnp.concatenate([edge_index_rel[1].astype(jnp.int32), trash])
    src_g = src.reshape(NCHT, CH)
    dst_g = jnp.where(dst >= N_SRL, 0, dst).reshape(NCHT, CH)
    st_row = jnp.concatenate(
        [span_start.astype(jnp.int32), zpad]).reshape(1, E_PAD)
    ln_row = jnp.concatenate(
        [span_len.astype(jnp.int32), zpad]).reshape(1, E_PAD)

    a_src, b_dst = _sc_gather(a_tab, b_tab, src_g, dst_g)

    ex, p = _tc_edge(a_src, b_dst, st_row, ln_row, g_tab,
                     nt_W1, b1, nt_W2, b2, aw1)

    zrows = jnp.zeros((ZPT, OUT), _f32)
    dst_s = dst.reshape(NS, NCH2, CH)
    denom, numer = _sc_scatter(ex, p, dst_s, zrows)

    h_srl = _tc_fin(denom, numer, feat_srl)

    s2_g = jnp.concatenate(
        [edge_index_s2t[0].astype(jnp.int32), zpad]).reshape(NCHT, CH)
    d2_g = jnp.concatenate(
        [edge_index_s2t[1].astype(jnp.int32), trash]).reshape(NCHT, CH)
    partials = _sc_s2t(h_srl, s2_g, d2_g, zrows)

    h_out = _tc_gru(partials[0], partials[1], h_tok, gru_Wih, gru_Whh, bih, bhh)
    return (h_srl, h_out)


# speedup vs baseline: 9.8896x; 1.2448x over previous
import functools

import jax
import jax.numpy as jnp
from jax import lax
from jax.experimental import pallas as pl
from jax.experimental.pallas import tpu as pltpu
from jax.experimental.pallas import tpu_sc as plsc

IN = 128
OUT = 128
BERT = 768
N_SRL = 10000
N_TOK = 10000
E_REL = 160000
E_S2T = 160000
T = 512
NSTART = 500
NLEN = 8
NSPAN = NLEN * NSTART

NC = 2
NS = 16
NW = NC * NS
CH = 128
E_PAD = 163840
EPW = E_PAD // NW
NCH = EPW // CH
EPT = E_PAD // NS
NCH2 = EPT // CH
ACC_N = 10240
ZPT = ACC_N // NS
CPA = 632
CPL = N_SRL - 15 * CPA

NCHT = E_PAD // CH
K0 = 56
K1 = (NCHT // NS) - K0
CB1 = NS * K0
NCHPT = NCHT // NS
GP = 512
HHALF = N_TOK // NC
HZERO = HHALF
HTAB = HHALF + 8
HSA = 312
HSL = HHALF - 15 * HSA

_f32 = jnp.float32


def _leaky(x):
    return jnp.where(x >= 0, x, 0.01 * x)


def _prep_body(bert_ref, fs_ref, ft_ref, rwa_ref, rwb_ref, rb_ref,
               w1_ref, b1_ref, w2_ref, b2_ref, aw2_ref, ab_ref,
               a_ref, r_ref, batt_ref, htok_ref):
    r_ref[...] = jnp.dot(bert_ref[...], rwb_ref[...],
                         preferred_element_type=_f32)

    def ntrans(x):
        u = jnp.dot(x, w1_ref[...], preferred_element_type=_f32) + b1_ref[...]
        u = _leaky(u)
        return jnp.dot(u, w2_ref[...], preferred_element_type=_f32) + b2_ref[...]

    fs = fs_ref[...]
    a_ref[...] = jnp.dot(fs, rwa_ref[...], preferred_element_type=_f32) + rb_ref[...]
    t = ntrans(fs)
    batt_ref[...] = jnp.dot(t, aw2_ref[...], preferred_element_type=_f32) + ab_ref[...]
    htok_ref[...] = ntrans(ft_ref[...])


def _tc_prep(bert, fs, ft, rwa, rwb, rb, w1, b1, w2, b2, aw2, ab):
    return pl.pallas_call(
        _prep_body,
        out_shape=(
            jax.ShapeDtypeStruct((N_SRL, IN), _f32),
            jax.ShapeDtypeStruct((GP, OUT), _f32),
            jax.ShapeDtypeStruct((N_SRL, OUT), _f32),
            jax.ShapeDtypeStruct((N_TOK, OUT), _f32),
        ),
    )(bert, fs, ft, rwa, rwb, rb, w1, b1, w2, b2, aw2, ab)


_MESH = plsc.VectorSubcoreMesh(core_axis_name="c", subcore_axis_name="s",
                               num_cores=NC, num_subcores=NS)


def _sc_gather(a_tab, b_tab, src_g, dst_g):
    @functools.partial(
        pl.kernel,
        out_type=(
            jax.ShapeDtypeStruct((E_PAD, OUT), _f32),
            jax.ShapeDtypeStruct((E_PAD, OUT), _f32),
        ),
        mesh=_MESH,
        scratch_types=[
            pltpu.VMEM((NCHPT, CH), jnp.int32),
            pltpu.VMEM((CH, OUT), _f32),
            pltpu.VMEM((CH, OUT), _f32),
            pltpu.VMEM_SHARED((N_SRL, OUT), _f32),
            pltpu.SemaphoreType.DMA,
            pltpu.SemaphoreType.DMA,
        ],
    )
    def k(a_h, b_h, src_h, dst_h, oa, ob, idx_v, r0, r1, tab, g0, g1):
        c = lax.axis_index("c")
        s = lax.axis_index("s")

        def stage(tab_h):
            @pl.when(s < NS - 1)
            def _():
                pltpu.sync_copy(tab_h.at[pl.ds(s * CPA, CPA)],
                                tab.at[pl.ds(s * CPA, CPA)])

            @pl.when(s == NS - 1)
            def _():
                pltpu.sync_copy(tab_h.at[pl.ds((NS - 1) * CPA, CPL)],
                                tab.at[pl.ds((NS - 1) * CPA, CPL)])

        def pipe(idx_h, out_h):
            gbase = s * NCHPT
            pltpu.sync_copy(idx_h.at[pl.ds(gbase, NCHPT)], idx_v)

            def gat(j, buf, sem):
                pltpu.async_copy(tab.at[idx_v.at[j]], buf, sem)

            def drain(buf, sem):
                pltpu.make_async_copy(a_h.at[pl.ds(0, CH)], buf, sem).wait()

            def write(j, buf):
                pltpu.sync_copy(buf, out_h.at[pl.ds((gbase + j) * CH, CH)])

            gat(0, r0, g0)

            def body(k_, carry):
                j0 = 2 * k_
                j1 = j0 + 1
                drain(r0, g0)
                gat(j1, r1, g1)
                write(j0, r0)
                drain(r1, g1)
                gat(jnp.minimum(j0 + 2, NCHPT - 1), r0, g0)
                write(j1, r1)
                return carry

            lax.fori_loop(0, NCHPT // 2, body, 0)
            drain(r0, g0)

        @pl.when(c == 0)
        def _():
            stage(a_h)

        @pl.when(c == 1)
        def _():
            stage(b_h)

        plsc.subcore_barrier()

        @pl.when(c == 0)
        def _():
            pipe(src_h, oa)

        @pl.when(c == 1)
        def _():
            pipe(dst_h, ob)

    return k(a_tab, b_tab, src_g, dst_g)


_BE = 2048


def _edge_body(a_ref, b_ref, st_ref, ln_ref, g_ref,
               w1_ref, b1_ref, w2_ref, b2_ref, aw1_ref, ex_ref, p_ref):
    st = st_ref[...]
    ln = ln_ref[...]
    yy = st + ln + 1
    inv = _f32(1.0) / (ln.astype(_f32) + _f32(1.0))
    tt = lax.broadcasted_iota(jnp.int32, (GP, _BE), 0)
    mt = jnp.where((tt >= st) & (tt < yy), inv, _f32(0.0))
    rel = lax.dot_general(mt, g_ref[...], (((0,), (0,)), ((), ())),
                          preferred_element_type=_f32)
    pre = a_ref[...] + rel
    u = jnp.dot(pre, w1_ref[...], preferred_element_type=_f32) + b1_ref[...]
    u = _leaky(u)
    m = jnp.dot(u, w2_ref[...], preferred_element_type=_f32) + b2_ref[...]
    e = jnp.dot(m, aw1_ref[...], preferred_element_type=_f32) + b_ref[...]
    e = _leaky(e)
    ex = jnp.exp(e)
    ex_ref[...] = ex
    p_ref[...] = ex * m


def _tc_edge(a_src, b_dst, st_row, ln_row, g_tab, w1, b1, w2, b2, aw1):
    nblk = E_PAD // _BE
    edge_spec = pl.BlockSpec((_BE, OUT), lambda i: (i, 0))
    row_spec = pl.BlockSpec((1, _BE), lambda i: (0, i))
    g_spec = pl.BlockSpec((GP, OUT), lambda i: (0, 0))
    w_spec = pl.BlockSpec((IN, OUT), lambda i: (0, 0))
    bias_spec = pl.BlockSpec((1, OUT), lambda i: (0, 0))
    return pl.pallas_call(
        _edge_body,
        grid=(nblk,),
        in_specs=[edge_spec, edge_spec, row_spec, row_spec, g_spec,
                  w_spec, bias_spec, w_spec, bias_spec, w_spec],
        out_specs=[edge_spec, edge_spec],
        out_shape=(
            jax.ShapeDtypeStruct((E_PAD, OUT), _f32),
            jax.ShapeDtypeStruct((E_PAD, OUT), _f32),
        ),
    )(a_src, b_dst, st_row, ln_row, g_tab, w1, b1, w2, b2, aw1)


def _sc_scatter(ex, p, dst_s, zrows):
    @functools.partial(
        pl.kernel,
        out_type=(
            jax.ShapeDtypeStruct((N_SRL, OUT), _f32),
            jax.ShapeDtypeStruct((N_SRL, OUT), _f32),
        ),
        mesh=_MESH,
        scratch_types=[
            pltpu.VMEM((NCH2, CH), jnp.int32),
            pltpu.VMEM((CH, OUT), _f32),
            pltpu.VMEM((CH, OUT), _f32),
            pltpu.VMEM_SHARED((ACC_N, OUT), _f32),
            pltpu.SemaphoreType.DMA,
            pltpu.SemaphoreType.DMA,
        ],
    )
    def k(ex_h, p_h, dst_h, z_h, den_o, num_o, idx_v, rows0, rows1, acc,
          r0, r1):
        c = lax.axis_index("c")
        s = lax.axis_index("s")
        pltpu.sync_copy(z_h, acc.at[pl.ds(s * ZPT, ZPT)])
        pltpu.sync_copy(dst_h.at[s], idx_v)
        plsc.subcore_barrier()

        def run(src_h):
            def read(j, buf, sem):
                pltpu.async_copy(src_h.at[pl.ds(s * EPT + j * CH, CH)],
                                 buf, sem)

            def drain(buf, sem):
                pltpu.make_async_copy(src_h.at[pl.ds(0, CH)], buf, sem).wait()

            def scat(j, buf):
                pltpu.sync_copy(buf, acc.at[idx_v.at[j]], add=True)

            read(0, rows0, r0)

            def body(k_, carry):
                j0 = 2 * k_
                j1 = j0 + 1
                drain(rows0, r0)
                read(j1, rows1, r1)
                scat(j0, rows0)
                drain(rows1, r1)
                read(jnp.minimum(j0 + 2, NCH2 - 1), rows0, r0)
                scat(j1, rows1)
                return carry

            lax.fori_loop(0, NCH2 // 2, body, 0)
            drain(rows0, r0)

        @pl.when(c == 0)
        def _():
            run(ex_h)

        @pl.when(c == 1)
        def _():
            run(p_h)

        plsc.subcore_barrier()

        @pl.when((c == 0) & (s < NS - 1))
        def _():
            pltpu.sync_copy(acc.at[pl.ds(s * CPA, CPA)],
                            den_o.at[pl.ds(s * CPA, CPA)])

        @pl.when((c == 0) & (s == NS - 1))
        def _():
            pltpu.sync_copy(acc.at[pl.ds((NS - 1) * CPA, CPL)],
                            den_o.at[pl.ds((NS - 1) * CPA, CPL)])

        @pl.when((c == 1) & (s < NS - 1))
        def _():
            pltpu.sync_copy(acc.at[pl.ds(s * CPA, CPA)],
                            num_o.at[pl.ds(s * CPA, CPA)])

        @pl.when((c == 1) & (s == NS - 1))
        def _():
            pltpu.sync_copy(acc.at[pl.ds((NS - 1) * CPA, CPL)],
                            num_o.at[pl.ds((NS - 1) * CPA, CPL)])

    return k(ex, p, dst_s, zrows)


def _fin_body(den_ref, num_ref, fs_ref, out_ref):
    den = den_ref[...]
    keep = den > 0
    safe = jnp.where(keep, den, _f32(1.0))
    out_ref[...] = jnp.where(keep, num_ref[...] / safe, fs_ref[...])


def _tc_fin(denom, numer, fs):
    return pl.pallas_call(
        _fin_body,
        out_shape=jax.ShapeDtypeStruct((N_SRL, IN), _f32),
    )(denom, numer, fs)


def _sc_gather2(h_srl, s2_g):
    @functools.partial(
        pl.kernel,
        out_type=jax.ShapeDtypeStruct((E_PAD, OUT), _f32),
        mesh=_MESH,
        scratch_types=[
            pltpu.VMEM((NCH, CH), jnp.int32),
            pltpu.VMEM((CH, OUT), _f32),
            pltpu.VMEM((CH, OUT), _f32),
            pltpu.VMEM_SHARED((N_TOK, OUT), _f32),
            pltpu.SemaphoreType.DMA,
            pltpu.SemaphoreType.DMA,
        ],
    )
    def k(h_h, s2_h, out_h, idx_v, r0, r1, tab, g0, g1):
        c = lax.axis_index("c")
        s = lax.axis_index("s")
        wid = s * NC + c

        @pl.when(s < NS - 1)
        def _():
            pltpu.sync_copy(h_h.at[pl.ds(s * CPA, CPA)],
                            tab.at[pl.ds(s * CPA, CPA)])

        @pl.when(s == NS - 1)
        def _():
            pltpu.sync_copy(h_h.at[pl.ds((NS - 1) * CPA, CPL)],
                            tab.at[pl.ds((NS - 1) * CPA, CPL)])

        gbase = wid * NCH
        pltpu.sync_copy(s2_h.at[pl.ds(gbase, NCH)], idx_v)
        plsc.subcore_barrier()

        def gat(j, buf, sem):
            pltpu.async_copy(tab.at[idx_v.at[j]], buf, sem)

        def drain(buf, sem):
            pltpu.make_async_copy(h_h.at[pl.ds(0, CH)], buf, sem).wait()

        def write(j, buf):
            pltpu.sync_copy(buf, out_h.at[pl.ds((gbase + j) * CH, CH)])

        gat(0, r0, g0)

        def body(k_, carry):
            j0 = 2 * k_
            j1 = j0 + 1
            drain(r0, g0)
            gat(j1, r1, g1)
            write(j0, r0)
            drain(r1, g1)
            gat(jnp.minimum(j0 + 2, NCH - 1), r0, g0)
            write(j1, r1)
            return carry

        lax.fori_loop(0, NCH // 2, body, 0)
        drain(r0, g0)

    return k(h_srl, s2_g)


def _sc_scatter2(hs2, d2_g, zrows):
    @functools.partial(
        pl.kernel,
        out_type=jax.ShapeDtypeStruct((NC, N_TOK, OUT), _f32),
        mesh=_MESH,
        scratch_types=[
            pltpu.VMEM((NCH, CH), jnp.int32),
            pltpu.VMEM((CH, OUT), _f32),
            pltpu.VMEM((CH, OUT), _f32),
            pltpu.VMEM_SHARED((ACC_N, OUT), _f32),
            pltpu.SemaphoreType.DMA,
            pltpu.SemaphoreType.DMA,
        ],
    )
    def k(v_h, d2_h, z_h, out_o, idx_v, r0, r1, acc, g0, g1):
        c = lax.axis_index("c")
        s = lax.axis_index("s")
        wid = s * NC + c
        pltpu.sync_copy(z_h, acc.at[pl.ds(s * ZPT, ZPT)])
        gbase = wid * NCH
        pltpu.sync_copy(d2_h.at[pl.ds(gbase, NCH)], idx_v)
        plsc.subcore_barrier()

        def read(j, buf, sem):
            pltpu.async_copy(v_h.at[pl.ds((gbase + j) * CH, CH)], buf, sem)

        def drain(buf, sem):
            pltpu.make_async_copy(v_h.at[pl.ds(0, CH)], buf, sem).wait()

        def scat(j, buf):
            pltpu.sync_copy(buf, acc.at[idx_v.at[j]], add=True)

        read(0, r0, g0)

        def body(k_, carry):
            j0 = 2 * k_
            j1 = j0 + 1
            drain(r0, g0)
            read(j1, r1, g1)
            scat(j0, r0)
            drain(r1, g1)
            read(jnp.minimum(j0 + 2, NCH - 1), r0, g0)
            scat(j1, r1)
            return carry

        lax.fori_loop(0, NCH // 2, body, 0)
        drain(r0, g0)
        plsc.subcore_barrier()

        @pl.when(s < NS - 1)
        def _():
            pltpu.sync_copy(acc.at[pl.ds(s * CPA, CPA)],
                            out_o.at[c, pl.ds(s * CPA, CPA)])

        @pl.when(s == NS - 1)
        def _():
            pltpu.sync_copy(acc.at[pl.ds((NS - 1) * CPA, CPL)],
                            out_o.at[c, pl.ds((NS - 1) * CPA, CPL)])

    return k(hs2, d2_g, zrows)


def _gru_body(p0_ref, p1_ref, htok_ref, wih_ref, whh_ref, bih_ref, bhh_ref,
              out_ref):
    x1 = p0_ref[...] + p1_ref[...]
    h = jnp.zeros((N_TOK, OUT), _f32)
    for x in (x1, htok_ref[...]):
        gi = jnp.dot(x, wih_ref[...], preferred_element_type=_f32) + bih_ref[...]
        gh = jnp.dot(h, whh_ref[...], preferred_element_type=_f32) + bhh_ref[...]
        ir, iz, inn = gi[:, :OUT], gi[:, OUT:2 * OUT], gi[:, 2 * OUT:]
        hr, hz, hn = gh[:, :OUT], gh[:, OUT:2 * OUT], gh[:, 2 * OUT:]
        r = jax.nn.sigmoid(ir + hr)
        z = jax.nn.sigmoid(iz + hz)
        n = jnp.tanh(inn + r * hn)
        h = (1.0 - z) * n + z * h
    out_ref[...] = h


def _tc_gru(p0, p1, htok, wih, whh, bih, bhh):
    return pl.pallas_call(
        _gru_body,
        out_shape=jax.ShapeDtypeStruct((N_TOK, OUT), _f32),
    )(p0, p1, htok, wih, whh, bih, bhh)


def kernel(feat_srl, feat_tok, bert_token_emb, edge_index_rel, span_start,
           span_len, edge_index_s2t, rel_W, rel_b, nt_W1, nt_b1, nt_W2, nt_b2,
           att_W, att_b, gru_Wih, gru_Whh, gru_bih, gru_bhh):
    rwa = rel_W[:IN]
    rwb = rel_W[IN:]
    aw1 = att_W[:OUT]
    aw2 = att_W[OUT:]
    rb = rel_b.reshape(1, IN)
    b1 = nt_b1.reshape(1, IN)
    b2 = nt_b2.reshape(1, OUT)
    ab = att_b.reshape(1, OUT)
    bih = gru_bih.reshape(1, 3 * OUT)
    bhh = gru_bhh.reshape(1, 3 * OUT)

    a_tab, g_tab, b_tab, h_tok = _tc_prep(
        bert_token_emb, feat_srl, feat_tok, rwa, rwb, rb,
        nt_W1, b1, nt_W2, b2, aw2, ab)

    npad = E_PAD - E_REL
    zpad = jnp.zeros((npad,), jnp.int32)
    trash = jnp.full((npad,), N_SRL, jnp.int32)
    src = jnp.concatenate([edge_index_rel[0].astype(jnp.int32), zpad])
    dst = jnp.concatenate([edge_index_rel[1].astype(jnp.int32), trash])
    src_g = src.reshape(NCHT, CH)
    dst_g = jnp.where(dst >= N_SRL, 0, dst).reshape(NCHT, CH)
    st_row = jnp.concatenate(
        [span_start.astype(jnp.int32), zpad]).reshape(1, E_PAD)
    ln_row = jnp.concatenate(
        [span_len.astype(jnp.int32), zpad]).reshape(1, E_PAD)

    a_src, b_dst = _sc_gather(a_tab, b_tab, src_g, dst_g)

    ex, p = _tc_edge(a_src, b_dst, st_row, ln_row, g_tab,
                     nt_W1, b1, nt_W2, b2, aw1)

    zrows = jnp.zeros((ZPT, OUT), _f32)
    dst_s = dst.reshape(NS, NCH2, CH)
    denom, numer = _sc_scatter(ex, p, dst_s, zrows)

    h_srl = _tc_fin(denom, numer, feat_srl)

    s2_g = jnp.concatenate(
        [edge_index_s2t[0].astype(jnp.int32), zpad]).reshape(NCHT, CH)
    d2_g = jnp.concatenate(
        [edge_index_s2t[1].astype(jnp.int32), trash]).reshape(NCHT, CH)
    hs2 = _sc_gather2(h_srl, s2_g)
    partials = _sc_scatter2(hs2, d2_g, zrows)

    h_out = _tc_gru(partials[0], partials[1], h_tok, gru_Wih, gru_Whh, bih, bhh)
    return (h_srl, h_out)
